# baseline (speedup 1.0000x reference)
import functools
import jax
import jax.numpy as jnp
from jax.experimental import pallas as pl
from jax.experimental.pallas import tpu as pltpu

N = 25600
E = 102400
B = 1024
D = 128
H = 2
C = 64
NB = 4
HID = 2048
OUT = 86


def _mm_body(xr, wr, br, o):
    o[...] = jnp.dot(xr[...], wr[...], preferred_element_type=jnp.float32) + br[...]


def _mm(x, w, b, bm=2048):
    M, K = x.shape
    Ko = w.shape[1]
    return pl.pallas_call(
        _mm_body,
        grid=(M // bm,),
        in_specs=[
            pl.BlockSpec((bm, K), lambda i: (i, 0)),
            pl.BlockSpec((K, Ko), lambda i: (0, 0)),
            pl.BlockSpec((1, Ko), lambda i: (0, 0)),
        ],
        out_specs=pl.BlockSpec((bm, Ko), lambda i: (i, 0)),
        out_shape=jax.ShapeDtypeStruct((M, Ko), jnp.float32),
    )(x, w, b.reshape(1, Ko))


def _mlp_layer_body(xr, wr, br, gr, ber, o):
    h = jnp.dot(xr[...], wr[...], preferred_element_type=jnp.float32) + br[...]
    mu = jnp.mean(h, axis=0, keepdims=True)
    xc = h - mu
    var = jnp.mean(xc * xc, axis=0, keepdims=True)
    h = xc / jnp.sqrt(var + 1e-5) * gr[...] + ber[...]
    o[...] = jnp.maximum(h, 0.0)


def _mlp_layer(x, w, b, g, be, bk=512):
    M, K = x.shape
    Ko = w.shape[1]
    return pl.pallas_call(
        _mlp_layer_body,
        grid=(Ko // bk,),
        in_specs=[
            pl.BlockSpec((M, K), lambda j: (0, 0)),
            pl.BlockSpec((K, bk), lambda j: (0, j)),
            pl.BlockSpec((1, bk), lambda j: (0, j)),
            pl.BlockSpec((1, bk), lambda j: (0, j)),
            pl.BlockSpec((1, bk), lambda j: (0, j)),
        ],
        out_specs=pl.BlockSpec((M, bk), lambda j: (0, j)),
        out_shape=jax.ShapeDtypeStruct((M, Ko), jnp.float32),
    )(x, w, b.reshape(1, Ko), g.reshape(1, Ko), be.reshape(1, Ko))


def _seg_softmax_nomax(s, seg, num):
    e = jnp.exp(s)
    z = jax.ops.segment_sum(e, seg, num_segments=num)
    return e / (z[seg] + 1e-16)


def _graph_ln(x, batch, num, w, b):
    f = x.shape[-1]
    cnt = jax.ops.segment_sum(jnp.ones((x.shape[0],), x.dtype), batch, num_segments=num) * f
    cnt = jnp.maximum(cnt, 1.0)
    mean = jax.ops.segment_sum(x.sum(-1), batch, num_segments=num) / cnt
    xc = x - mean[batch][:, None]
    var = jax.ops.segment_sum((xc * xc).sum(-1), batch, num_segments=num) / cnt
    out = xc / jnp.sqrt(var + 1e-5)[batch][:, None]
    return out * w + b


def _normalize(x):
    n = jnp.linalg.norm(x, axis=-1, keepdims=True)
    return x / jnp.maximum(n, 1e-12)


def kernel(x1, edge_index1, x1_batch, x2, edge_index2, x2_batch, ln0_w, ln0_b,
           gat_Wl, gat_bl, gat_Wr, gat_br, gat_att, gat_bias, sag_Wrel, sag_brel,
           sag_Wroot, nn_w, nn_b, ca_wq, ca_wk, ca_bias, ca_a, mlp_Wh, mlp_bh,
           mlp_g, mlp_be, mlp_Wo, mlp_bo):
    x = jnp.concatenate([x1, x2], axis=0)
    src = jnp.concatenate([edge_index1[0], edge_index2[0] + N])
    dst = jnp.concatenate([edge_index1[1], edge_index2[1] + N])
    batch = jnp.concatenate([x1_batch, x2_batch + B])
    NT, BT = 2 * N, 2 * B

    x = _graph_ln(x, batch, BT, ln0_w, ln0_b)

    embs = []
    for i in range(NB):
        wlr = jnp.concatenate([gat_Wl[i], gat_Wr[i]], axis=1)
        blr = jnp.concatenate([gat_bl[i], gat_br[i]], axis=0)
        xlr = _mm(x, wlr, blr)
        xl = xlr[:, :D].reshape(NT, H, C)
        xr = xlr[:, D:].reshape(NT, H, C)

        e = jax.nn.leaky_relu(xl[src] + xr[dst], 0.2)
        logit = (e * gat_att[i][None]).sum(-1)
        ee = jnp.exp(logit)
        z = jax.ops.segment_sum(ee, dst, num_segments=NT)
        num = jax.ops.segment_sum(xl[src] * ee[..., None], dst, num_segments=NT)
        xg = (num / (z[..., None] + 1e-16)).reshape(NT, D) + gat_bias[i]

        wrr = jnp.concatenate([sag_Wrel[i], sag_Wroot[i]], axis=1)
        pad = jnp.zeros((D, 126), jnp.float32)
        xw = _mm(xg, jnp.concatenate([wrr, pad], axis=1), jnp.zeros((D,), jnp.float32))
        score = jax.ops.segment_sum(xw[:, 0][src], dst, num_segments=NT) + sag_brel[i][0] + xw[:, 1]
        s = _seg_softmax_nomax(score, batch, BT)
        emb = jax.ops.segment_sum(xg * s[:, None], batch, num_segments=BT)
        embs.append(emb)

        x = jax.nn.elu(_graph_ln(xg, batch, BT, nn_w[i], nn_b[i]))

    emb = jnp.stack(embs, axis=1)
    repr1, repr2 = emb[:B], emb[B:]

    keys = repr1 @ ca_wk
    queries = repr2 @ ca_wq
    e_act = queries[:, None, :, :] + keys[:, :, None, :] + ca_bias
    atts = jnp.tanh(e_act) @ ca_a
    r1n = _normalize(repr1)
    r2n = _normalize(repr2)
    rws = []
    for i in range(NB):
        for j in range(NB):
            rws.append((r1n[:, i] + r2n[:, j]) * atts[:, i, j][:, None])
    h = jnp.concatenate(rws, axis=-1)

    for l in range(3):
        h = _mlp_layer(h, mlp_Wh[l], mlp_bh[l], mlp_g[l], mlp_be[l])

    wo = jnp.concatenate([mlp_Wo, jnp.zeros((HID, 128 - OUT), jnp.float32)], axis=1)
    bo = jnp.concatenate([mlp_bo, jnp.zeros((128 - OUT,), jnp.float32)])
    out = _mm(h, wo, bo, bm=1024)
    return out[:, :OUT]


# speedup vs baseline: 13.9366x; 13.9366x over previous
import functools
import jax
import jax.numpy as jnp
from jax import lax
from jax.experimental import pallas as pl
from jax.experimental.pallas import tpu as pltpu
from jax.experimental.pallas import tpu_sc as plsc

N = 25600
E = 102400
B = 1024
D = 128
H = 2
C = 64
NB = 4
HID = 2048
OUT = 86

N2 = 2 * N
B2 = 2 * B

NPT = N // 16
EPT = E // 16
GPT = B // 16
CH = 128
NCH = N // CH
NIT = 13
ECH = EPT // CH

_MESH = plsc.VectorSubcoreMesh(core_axis_name="c", subcore_axis_name="s")


def _lane_masks():
    lanes = lax.iota(jnp.int32, 16)
    return [lanes == j for j in range(16)]


def _ins(acc, mask, scalar):
    return jnp.where(mask, jnp.full((16,), scalar, jnp.float32), acc)


def _hsum(v):
    parts = [v[j] for j in range(16)]
    while len(parts) > 1:
        parts = [parts[i] + parts[i + 1] for i in range(0, len(parts), 2)]
    return parts[0]


def _rsqrt_newton(x):
    i = lax.bitcast_convert_type(x, jnp.int32)
    i = jnp.int32(0x5F3759DF) - lax.shift_right_logical(i, jnp.int32(1))
    y = lax.bitcast_convert_type(i, jnp.float32)
    for _ in range(4):
        y = y * (1.5 - 0.5 * x * y * y)
    return y


def _zero2d(ref, rows, cols):
    z = jnp.zeros((16,), jnp.float32)

    def rbody(r, carry):
        for q in range(cols // 16):
            ref[r, pl.ds(16 * q, 16)] = z
        return carry

    lax.fori_loop(0, rows, rbody, 0)


def _fill1d(ref, n, val):
    v = jnp.full((16,), val, jnp.float32)

    def kbody(k, carry):
        ref[pl.ds(16 * k, 16)] = v
        return carry

    lax.fori_loop(0, n // 16, kbody, 0)


@functools.partial(
    pl.kernel,
    out_type=jax.ShapeDtypeStruct((N2, D), jnp.float32),
    mesh=_MESH,
    scratch_types=[
        pltpu.VMEM_SHARED((B,), jnp.float32),
        pltpu.VMEM_SHARED((B,), jnp.float32),
        pltpu.VMEM_SHARED((B,), jnp.float32),
        pltpu.VMEM((CH, D), jnp.float32),
        pltpu.VMEM((CH,), jnp.float32),
        pltpu.VMEM((CH,), jnp.float32),
        pltpu.VMEM((CH,), jnp.float32),
        pltpu.VMEM((CH,), jnp.float32),
        pltpu.VMEM((CH,), jnp.float32),
        pltpu.VMEM((CH,), jnp.int32),
        pltpu.VMEM((8, D), jnp.float32),
        pltpu.SemaphoreType.DMA,
    ],
)
def _sc_ln0(x_hbm, batch_hbm, wb_hbm, out_hbm, sum_sh, sq_sh, cnt_sh,
            xbuf, sumb, sqb, onesb, meanb, rstdb, bidx, pbuf, sem):
    c = lax.axis_index("c")
    s = lax.axis_index("s")
    masks = _lane_masks()
    _fill1d(sumb, CH, 0.0)
    _fill1d(onesb, CH, 1.0)
    pltpu.sync_copy(sumb.at[pl.ds(0, GPT)], sum_sh.at[pl.ds(s * GPT, GPT)])
    pltpu.sync_copy(sumb.at[pl.ds(0, GPT)], sq_sh.at[pl.ds(s * GPT, GPT)])
    pltpu.sync_copy(sumb.at[pl.ds(0, GPT)], cnt_sh.at[pl.ds(s * GPT, GPT)])
    pltpu.sync_copy(wb_hbm, pbuf)
    plsc.subcore_barrier()

    def p1(i, carry):
        chunk = s + 16 * i

        @pl.when(chunk < NCH)
        def _():
            base = chunk * CH
            gbase = c * N + base
            pltpu.sync_copy(x_hbm.at[pl.ds(gbase, CH)], xbuf)
            pltpu.sync_copy(batch_hbm.at[c, pl.ds(base, CH)], bidx)

            def grp(k, carry2):
                sumv = jnp.zeros((16,), jnp.float32)
                sqv = jnp.zeros((16,), jnp.float32)
                for j in range(16):
                    r = 16 * k + j
                    sv = jnp.zeros((16,), jnp.float32)
                    qv = jnp.zeros((16,), jnp.float32)
                    for q in range(D // 16):
                        v = xbuf[r, pl.ds(16 * q, 16)]
                        sv = sv + v
                        qv = qv + v * v
                    sumv = _ins(sumv, masks[j], _hsum(sv))
                    sqv = _ins(sqv, masks[j], _hsum(qv))
                sumb[pl.ds(16 * k, 16)] = sumv
                sqb[pl.ds(16 * k, 16)] = sqv
                return carry2

            lax.fori_loop(0, CH // 16, grp, 0)
            pltpu.sync_copy(sumb, sum_sh.at[bidx], add=True)
            pltpu.sync_copy(sqb, sq_sh.at[bidx], add=True)
            pltpu.sync_copy(onesb, cnt_sh.at[bidx], add=True)

        return carry

    lax.fori_loop(0, NIT, p1, 0)
    plsc.subcore_barrier()

    pltpu.sync_copy(sum_sh.at[pl.ds(s * GPT, GPT)], sumb.at[pl.ds(0, GPT)])
    pltpu.sync_copy(sq_sh.at[pl.ds(s * GPT, GPT)], sqb.at[pl.ds(0, GPT)])
    pltpu.sync_copy(cnt_sh.at[pl.ds(s * GPT, GPT)], onesb.at[pl.ds(0, GPT)])
    for g in range(GPT // 16):
        sl = pl.ds(16 * g, 16)
        cf = jnp.maximum(onesb[sl] * jnp.float32(D), 1.0)
        mean = sumb[sl] / cf
        var = jnp.maximum(sqb[sl] / cf - mean * mean, 0.0) + 1e-5
        sumb[sl] = mean
        sqb[sl] = _rsqrt_newton(var)
    pltpu.sync_copy(sumb.at[pl.ds(0, GPT)], sum_sh.at[pl.ds(s * GPT, GPT)])
    pltpu.sync_copy(sqb.at[pl.ds(0, GPT)], sq_sh.at[pl.ds(s * GPT, GPT)])
    plsc.subcore_barrier()

    def p3(i, carry):
        chunk = s + 16 * i

        @pl.when(chunk < NCH)
        def _():
            base = chunk * CH
            gbase = c * N + base
            pltpu.sync_copy(x_hbm.at[pl.ds(gbase, CH)], xbuf)
            pltpu.sync_copy(batch_hbm.at[c, pl.ds(base, CH)], bidx)
            pltpu.async_copy(sum_sh.at[bidx], meanb, sem).wait()
            pltpu.async_copy(sq_sh.at[bidx], rstdb, sem).wait()

            def rgrp(k, carry2):
                mv = meanb[pl.ds(16 * k, 16)]
                rv = rstdb[pl.ds(16 * k, 16)]
                for j in range(16):
                    r = 16 * k + j
                    mean = mv[j]
                    rstd = rv[j]
                    for q in range(D // 16):
                        v = xbuf[r, pl.ds(16 * q, 16)]
                        xbuf[r, pl.ds(16 * q, 16)] = (v - mean) * rstd * pbuf[0, pl.ds(16 * q, 16)] + pbuf[1, pl.ds(16 * q, 16)]
                return carry2

            lax.fori_loop(0, CH // 16, rgrp, 0)
            pltpu.sync_copy(xbuf, out_hbm.at[pl.ds(gbase, CH)])

        return carry

    lax.fori_loop(0, NIT, p3, 0)


NQ = 4
NH = N // NQ
NHP = NH + 8
HPT = NH // 16
ZCH = NH // CH

@functools.partial(
    pl.kernel,
    out_type=[jax.ShapeDtypeStruct((N2, D), jnp.float32),
              jax.ShapeDtypeStruct((H, N2 // CH, CH), jnp.float32)],
    mesh=_MESH,
    scratch_types=[
        pltpu.VMEM_SHARED((NHP, D), jnp.float32),
        pltpu.VMEM_SHARED((NHP,), jnp.float32),
        pltpu.VMEM_SHARED((NHP,), jnp.float32),
        pltpu.VMEM((80, D), jnp.float32),
        pltpu.VMEM((80,), jnp.float32),
        pltpu.VMEM((CH, D), jnp.float32),
        pltpu.VMEM((CH, D), jnp.float32),
        pltpu.VMEM((CH,), jnp.float32),
        pltpu.VMEM((CH,), jnp.float32),
        pltpu.VMEM((2 * EPT,), jnp.float32),
        pltpu.VMEM((CH,), jnp.int32),
        pltpu.VMEM((CH,), jnp.int32),
        pltpu.VMEM((CH,), jnp.int32),
        pltpu.VMEM((8, D), jnp.float32),
        pltpu.SemaphoreType.DMA,
    ],
)
def _sc_edge(xl_hbm, xr_hbm, srcg_hbm, dstg_hbm, dstl_hbm,
             att_hbm, out_hbm, z_hbm, num_sh, z0_sh, z1_sh, zbuf, zer1,
             gl, gr, ez0, ez1, lbuf, isg, idg, idl, attv, sem):
    c = lax.axis_index("c")
    s = lax.axis_index("s")
    masks = _lane_masks()
    _zero2d(zbuf, 80, D)
    _fill1d(zer1, 80, 0.0)
    pltpu.sync_copy(att_hbm, attv)

    for p in range(NQ):
        rb = s * HPT
        for k in range(HPT // 80):
            pltpu.sync_copy(zbuf, num_sh.at[pl.ds(rb + k * 80, 80)])
            pltpu.sync_copy(zer1, z0_sh.at[pl.ds(rb + k * 80, 80)])
            pltpu.sync_copy(zer1, z1_sh.at[pl.ds(rb + k * 80, 80)])

        @pl.when(s == 15)
        def _():
            pltpu.sync_copy(zbuf.at[pl.ds(0, 8)], num_sh.at[pl.ds(NH, 8)])
            pltpu.sync_copy(zer1.at[pl.ds(0, 8)], z0_sh.at[pl.ds(NH, 8)])
            pltpu.sync_copy(zer1.at[pl.ds(0, 8)], z1_sh.at[pl.ds(NH, 8)])

        plsc.subcore_barrier()

        def edge_chunk(it, carry):
            base = s * EPT + it * CH
            lb = it * CH
            pltpu.sync_copy(srcg_hbm.at[c, pl.ds(base, CH)], isg)
            pltpu.sync_copy(dstl_hbm.at[c, pl.ds(base, CH)], idl)
            pltpu.async_copy(xl_hbm.at[isg], gl, sem).wait()

            def idxk(k, carry2):
                sl = pl.ds(16 * k, 16)
                loc = idl[sl] - jnp.int32(p * NH)
                ok = (loc >= 0) & (loc < NH)
                idl[sl] = jnp.where(ok, loc, jnp.int32(NH))
                return carry2

            lax.fori_loop(0, CH // 16, idxk, 0)

            if p == 0:
                pltpu.sync_copy(dstg_hbm.at[c, pl.ds(base, CH)], idg)
                pltpu.async_copy(xr_hbm.at[idg], gr, sem).wait()

                def grp(k, carry2):
                    lv0 = jnp.zeros((16,), jnp.float32)
                    lv1 = jnp.zeros((16,), jnp.float32)
                    for j in range(16):
                        e = 16 * k + j
                        acc0 = jnp.zeros((16,), jnp.float32)
                        acc1 = jnp.zeros((16,), jnp.float32)
                        for q in range(4):
                            a = gl[e, pl.ds(16 * q, 16)]
                            b = gr[e, pl.ds(16 * q, 16)]
                            t = a + b
                            tl = jnp.maximum(t, 0.2 * t)
                            acc0 = acc0 + tl * attv[0, pl.ds(16 * q, 16)]
                            a1 = gl[e, pl.ds(64 + 16 * q, 16)]
                            b1 = gr[e, pl.ds(64 + 16 * q, 16)]
                            t1 = a1 + b1
                            tl1 = jnp.maximum(t1, 0.2 * t1)
                            acc1 = acc1 + tl1 * attv[1, pl.ds(16 * q, 16)]
                        lv0 = _ins(lv0, masks[j], _hsum(acc0))
                        lv1 = _ins(lv1, masks[j], _hsum(acc1))
                    e0 = jnp.exp(lv0)
                    e1 = jnp.exp(lv1)
                    ez0[pl.ds(16 * k, 16)] = e0
                    ez1[pl.ds(16 * k, 16)] = e1
                    lbuf[pl.ds(lb + 16 * k, 16)] = e0
                    lbuf[pl.ds(EPT + lb + 16 * k, 16)] = e1
                    return carry2

                lax.fori_loop(0, CH // 16, grp, 0)
            else:
                def cpy(k, carry2):
                    ez0[pl.ds(16 * k, 16)] = lbuf[pl.ds(lb + 16 * k, 16)]
                    ez1[pl.ds(16 * k, 16)] = lbuf[pl.ds(EPT + lb + 16 * k, 16)]
                    return carry2

                lax.fori_loop(0, CH // 16, cpy, 0)

            def wgt(k, carry2):
                sv0 = ez0[pl.ds(16 * k, 16)]
                sv1 = ez1[pl.ds(16 * k, 16)]
                for j in range(16):
                    e = 16 * k + j
                    sc0 = sv0[j]
                    sc1 = sv1[j]
                    for q in range(4):
                        gl[e, pl.ds(16 * q, 16)] = gl[e, pl.ds(16 * q, 16)] * sc0
                        gl[e, pl.ds(64 + 16 * q, 16)] = gl[e, pl.ds(64 + 16 * q, 16)] * sc1
                return carry2

            lax.fori_loop(0, CH // 16, wgt, 0)
            pltpu.sync_copy(gl, num_sh.at[idl], add=True)
            pltpu.sync_copy(ez0, z0_sh.at[idl], add=True)
            pltpu.sync_copy(ez1, z1_sh.at[idl], add=True)
            return carry

        lax.fori_loop(0, ECH, edge_chunk, 0)
        plsc.subcore_barrier()

        pltpu.sync_copy(num_sh.at[pl.ds(s * HPT, HPT)],
                        out_hbm.at[pl.ds(c * N + p * NH + s * HPT, HPT)])

        def zflush(i, carry):
            chunk = s + 16 * i

            @pl.when(chunk < ZCH)
            def _():
                gchunk = c * (N // CH) + p * ZCH + chunk
                pltpu.sync_copy(z0_sh.at[pl.ds(chunk * CH, CH)], z_hbm.at[0, gchunk])
                pltpu.sync_copy(z1_sh.at[pl.ds(chunk * CH, CH)], z_hbm.at[1, gchunk])

            return carry

        lax.fori_loop(0, 4, zflush, 0)
        if p < NQ - 1:
            plsc.subcore_barrier()


@functools.partial(
    pl.kernel,
    out_type=[jax.ShapeDtypeStruct((N2, D), jnp.float32),
              jax.ShapeDtypeStruct((B2, D), jnp.float32)],
    mesh=_MESH,
    scratch_types=[
        pltpu.VMEM_SHARED((N,), jnp.float32),
        pltpu.VMEM_SHARED((N,), jnp.float32),
        pltpu.VMEM_SHARED((B,), jnp.float32),
        pltpu.VMEM_SHARED((B,), jnp.float32),
        pltpu.VMEM_SHARED((B,), jnp.float32),
        pltpu.VMEM_SHARED((B,), jnp.float32),
        pltpu.VMEM_SHARED((B, D), jnp.float32),
        pltpu.VMEM((CH, D), jnp.float32),
        pltpu.VMEM((CH, D), jnp.float32),
        pltpu.VMEM((8, D), jnp.float32),
        pltpu.VMEM((CH,), jnp.float32),
        pltpu.VMEM((CH,), jnp.float32),
        pltpu.VMEM((CH,), jnp.float32),
        pltpu.VMEM((CH,), jnp.float32),
        pltpu.VMEM((CH,), jnp.float32),
        pltpu.VMEM((CH,), jnp.float32),
        pltpu.VMEM((CH,), jnp.float32),
        pltpu.VMEM((CH,), jnp.float32),
        pltpu.VMEM((CH,), jnp.float32),
        pltpu.VMEM((CH,), jnp.int32),
        pltpu.VMEM((CH,), jnp.int32),
        pltpu.VMEM((CH,), jnp.int32),
        pltpu.VMEM((CH,), jnp.float32),
        pltpu.SemaphoreType.DMA,
    ],
)
def _sc_readout(xgh_hbm, z_hbm, srcl_hbm, dstl_hbm, batch_hbm, par_hbm,
                xnext_hbm, emb_hbm,
                xw_sh, sc_sh, zb_sh, sum_sh, sq_sh, cnt_sh, emb_sh,
                xbuf, wbuf, pbuf, sumb, sqb, onesb, xwbuf, scbuf, sbuf,
                zbv, z0ch, z1ch, bidx, isrc, idst, ebuf, sem):
    c = lax.axis_index("c")
    s = lax.axis_index("s")
    masks = _lane_masks()
    _zero2d(wbuf, CH, D)
    _fill1d(sumb, CH, 0.0)
    _fill1d(onesb, CH, 1.0)
    pltpu.sync_copy(sumb.at[pl.ds(0, GPT)], sum_sh.at[pl.ds(s * GPT, GPT)])
    pltpu.sync_copy(sumb.at[pl.ds(0, GPT)], sq_sh.at[pl.ds(s * GPT, GPT)])
    pltpu.sync_copy(sumb.at[pl.ds(0, GPT)], cnt_sh.at[pl.ds(s * GPT, GPT)])
    pltpu.sync_copy(sumb.at[pl.ds(0, GPT)], zb_sh.at[pl.ds(s * GPT, GPT)])
    pltpu.sync_copy(wbuf.at[pl.ds(0, GPT)], emb_sh.at[pl.ds(s * GPT, GPT)])
    pltpu.sync_copy(par_hbm, pbuf)
    plsc.subcore_barrier()

    def p1(i, carry):
        chunk = s + 16 * i

        @pl.when(chunk < NCH)
        def _():
            base = chunk * CH
            gbase = c * N + base
            pltpu.sync_copy(xgh_hbm.at[pl.ds(gbase, CH)], xbuf)
            pltpu.sync_copy(batch_hbm.at[c, pl.ds(base, CH)], bidx)
            pltpu.sync_copy(z_hbm.at[0, c * NCH + chunk], z0ch)
            pltpu.sync_copy(z_hbm.at[1, c * NCH + chunk], z1ch)
            brel = pbuf[5, pl.ds(0, 16)][0]

            def grp(k, carry2):
                sl = pl.ds(16 * k, 16)
                zv0 = 1.0 / (z0ch[sl] + 1e-16)
                zv1 = 1.0 / (z1ch[sl] + 1e-16)
                sumv = jnp.zeros((16,), jnp.float32)
                sqv = jnp.zeros((16,), jnp.float32)
                wrv = jnp.zeros((16,), jnp.float32)
                wtv = jnp.zeros((16,), jnp.float32)
                for j in range(16):
                    r = 16 * k + j
                    zi0 = zv0[j]
                    zi1 = zv1[j]
                    sv = jnp.zeros((16,), jnp.float32)
                    qv = jnp.zeros((16,), jnp.float32)
                    wr = jnp.zeros((16,), jnp.float32)
                    wt = jnp.zeros((16,), jnp.float32)
                    for q in range(D // 16):
                        zi = zi0 if q < 4 else zi1
                        v = xbuf[r, pl.ds(16 * q, 16)] * zi + pbuf[2, pl.ds(16 * q, 16)]
                        sv = sv + v
                        qv = qv + v * v
                        wr = wr + v * pbuf[0, pl.ds(16 * q, 16)]
                        wt = wt + v * pbuf[1, pl.ds(16 * q, 16)]
                    sumv = _ins(sumv, masks[j], _hsum(sv))
                    sqv = _ins(sqv, masks[j], _hsum(qv))
                    wrv = _ins(wrv, masks[j], _hsum(wr))
                    wtv = _ins(wtv, masks[j], _hsum(wt))
                sumb[pl.ds(16 * k, 16)] = sumv
                sqb[pl.ds(16 * k, 16)] = sqv
                xwbuf[pl.ds(16 * k, 16)] = wrv
                scbuf[pl.ds(16 * k, 16)] = wtv + brel
                return carry2

            lax.fori_loop(0, CH // 16, grp, 0)
            pltpu.sync_copy(xwbuf, xw_sh.at[pl.ds(base, CH)])
            pltpu.sync_copy(scbuf, sc_sh.at[pl.ds(base, CH)])
            pltpu.sync_copy(sumb, sum_sh.at[bidx], add=True)
            pltpu.sync_copy(sqb, sq_sh.at[bidx], add=True)
            pltpu.sync_copy(onesb, cnt_sh.at[bidx], add=True)

        return carry

    lax.fori_loop(0, NIT, p1, 0)
    plsc.subcore_barrier()

    def p2(it, carry):
        base = s * EPT + it * CH
        pltpu.sync_copy(srcl_hbm.at[c, pl.ds(base, CH)], isrc)
        pltpu.sync_copy(dstl_hbm.at[c, pl.ds(base, CH)], idst)
        pltpu.async_copy(xw_sh.at[isrc], ebuf, sem).wait()
        pltpu.sync_copy(ebuf, sc_sh.at[idst], add=True)
        return carry

    lax.fori_loop(0, ECH, p2, 0)
    plsc.subcore_barrier()

    def p3(i, carry):
        chunk = s + 16 * i

        @pl.when(chunk < NCH)
        def _():
            base = chunk * CH
            pltpu.sync_copy(sc_sh.at[pl.ds(base, CH)], sbuf)

            def expk(k, carry2):
                sbuf[pl.ds(16 * k, 16)] = jnp.exp(sbuf[pl.ds(16 * k, 16)])
                return carry2

            lax.fori_loop(0, CH // 16, expk, 0)
            pltpu.sync_copy(sbuf, sc_sh.at[pl.ds(base, CH)])
            pltpu.sync_copy(batch_hbm.at[c, pl.ds(base, CH)], bidx)
            pltpu.sync_copy(sbuf, zb_sh.at[bidx], add=True)

        return carry

    lax.fori_loop(0, NIT, p3, 0)
    plsc.subcore_barrier()

    pltpu.sync_copy(sum_sh.at[pl.ds(s * GPT, GPT)], sumb.at[pl.ds(0, GPT)])
    pltpu.sync_copy(sq_sh.at[pl.ds(s * GPT, GPT)], sqb.at[pl.ds(0, GPT)])
    pltpu.sync_copy(cnt_sh.at[pl.ds(s * GPT, GPT)], onesb.at[pl.ds(0, GPT)])
    for g in range(GPT // 16):
        sl = pl.ds(16 * g, 16)
        cf = jnp.maximum(onesb[sl] * jnp.float32(D), 1.0)
        mean = sumb[sl] / cf
        var = jnp.maximum(sqb[sl] / cf - mean * mean, 0.0) + 1e-5
        sumb[sl] = mean
        sqb[sl] = _rsqrt_newton(var)
    pltpu.sync_copy(sumb.at[pl.ds(0, GPT)], sum_sh.at[pl.ds(s * GPT, GPT)])
    pltpu.sync_copy(sqb.at[pl.ds(0, GPT)], sq_sh.at[pl.ds(s * GPT, GPT)])
    plsc.subcore_barrier()

    def p4(i, carry):
        chunk = s + 16 * i

        @pl.when(chunk < NCH)
        def _():
            base = chunk * CH
            gbase = c * N + base
            pltpu.sync_copy(xgh_hbm.at[pl.ds(gbase, CH)], xbuf)
            pltpu.sync_copy(batch_hbm.at[c, pl.ds(base, CH)], bidx)
            pltpu.sync_copy(sc_sh.at[pl.ds(base, CH)], sbuf)
            pltpu.sync_copy(z_hbm.at[0, c * NCH + chunk], z0ch)
            pltpu.sync_copy(z_hbm.at[1, c * NCH + chunk], z1ch)
            pltpu.async_copy(zb_sh.at[bidx], zbv, sem).wait()
            pltpu.async_copy(sum_sh.at[bidx], xwbuf, sem).wait()
            pltpu.async_copy(sq_sh.at[bidx], scbuf, sem).wait()

            def rgrp(k, carry2):
                sl = pl.ds(16 * k, 16)
                srv = sbuf[sl] / (zbv[sl] + 1e-16)
                zv0 = 1.0 / (z0ch[sl] + 1e-16)
                zv1 = 1.0 / (z1ch[sl] + 1e-16)
                mv = xwbuf[sl]
                rv = scbuf[sl]
                for j in range(16):
                    r = 16 * k + j
                    sr = srv[j]
                    zi0 = zv0[j]
                    zi1 = zv1[j]
                    mean = mv[j]
                    rstd = rv[j]
                    for q in range(D // 16):
                        zi = zi0 if q < 4 else zi1
                        v = xbuf[r, pl.ds(16 * q, 16)] * zi + pbuf[2, pl.ds(16 * q, 16)]
                        wbuf[r, pl.ds(16 * q, 16)] = v * sr
                        ln = (v - mean) * rstd * pbuf[3, pl.ds(16 * q, 16)] + pbuf[4, pl.ds(16 * q, 16)]
                        xbuf[r, pl.ds(16 * q, 16)] = jnp.where(ln > 0, ln, jnp.exp(ln) - 1.0)
                return carry2

            lax.fori_loop(0, CH // 16, rgrp, 0)
            pltpu.sync_copy(wbuf, emb_sh.at[bidx], add=True)
            pltpu.sync_copy(xbuf, xnext_hbm.at[pl.ds(gbase, CH)])

        return carry

    lax.fori_loop(0, NIT, p4, 0)
    plsc.subcore_barrier()
    pltpu.sync_copy(emb_sh.at[pl.ds(s * GPT, GPT)],
                    emb_hbm.at[pl.ds(c * B + s * GPT, GPT)])


def _mm2_body(xr, wr, br, o0, o1):
    y = jnp.dot(xr[...], wr[...], preferred_element_type=jnp.float32) + br[...]
    o0[...] = y[:, 0:128]
    o1[...] = y[:, 128:256]


def _mm2(x, w, b, bm=2048):
    M, K = x.shape
    outs = [jax.ShapeDtypeStruct((M, D), jnp.float32) for _ in range(2)]
    return pl.pallas_call(
        _mm2_body,
        grid=(M // bm,),
        in_specs=[
            pl.BlockSpec((bm, K), lambda i: (i, 0)),
            pl.BlockSpec((K, 2 * D), lambda i: (0, 0)),
            pl.BlockSpec((1, 2 * D), lambda i: (0, 0)),
        ],
        out_specs=[pl.BlockSpec((bm, D), lambda i: (i, 0)) for _ in range(2)],
        out_shape=outs,
    )(x, w, b.reshape(1, 2 * D))


def _coattn_body(e1, e2, wq, wk, cb, ca, o):
    r1 = [e1[i] for i in range(NB)]
    r2 = [e2[i] for i in range(NB)]
    keys = [jnp.dot(r, wk[...], preferred_element_type=jnp.float32) for r in r1]
    qrys = [jnp.dot(r, wq[...], preferred_element_type=jnp.float32) for r in r2]
    bias = cb[...]
    av = ca[...].reshape(D // 2, 1)
    r1n = [r / jnp.maximum(jnp.sqrt(jnp.sum(r * r, axis=1, keepdims=True)), 1e-12) for r in r1]
    r2n = [r / jnp.maximum(jnp.sqrt(jnp.sum(r * r, axis=1, keepdims=True)), 1e-12) for r in r2]
    for i in range(NB):
        for j in range(NB):
            att = jnp.dot(jnp.tanh(qrys[j] + keys[i] + bias), av,
                          preferred_element_type=jnp.float32)
            o[:, pl.ds(D * (NB * i + j), D)] = (r1n[i] + r2n[j]) * att


def _coattn(embs, wq, wk, cb, ca):
    return pl.pallas_call(
        _coattn_body,
        grid=(1,),
        in_specs=[
            pl.BlockSpec((NB, B, D), lambda i: (0, 0, 0)),
            pl.BlockSpec((NB, B, D), lambda i: (0, 1, 0)),
            pl.BlockSpec((D, D // 2), lambda i: (0, 0)),
            pl.BlockSpec((D, D // 2), lambda i: (0, 0)),
            pl.BlockSpec((1, D // 2), lambda i: (0, 0)),
            pl.BlockSpec((1, D // 2), lambda i: (0, 0)),
        ],
        out_specs=pl.BlockSpec((B, HID), lambda i: (0, 0)),
        out_shape=jax.ShapeDtypeStruct((B, HID), jnp.float32),
    )(embs, embs, wq, wk, cb.reshape(1, D // 2), ca.reshape(1, D // 2))


def _mlp_layer_body(xr, wr, br, gr, ber, o):
    h = jnp.dot(xr[...], wr[...], preferred_element_type=jnp.float32) + br[...]
    mu = jnp.mean(h, axis=0, keepdims=True)
    xc = h - mu
    var = jnp.mean(xc * xc, axis=0, keepdims=True)
    h = xc / jnp.sqrt(var + 1e-5) * gr[...] + ber[...]
    o[...] = jnp.maximum(h, 0.0)


def _mlp_layer(x, w, b, g, be, bk=512):
    M, K = x.shape
    Ko = w.shape[1]
    return pl.pallas_call(
        _mlp_layer_body,
        grid=(Ko // bk,),
        in_specs=[
            pl.BlockSpec((M, K), lambda j: (0, 0)),
            pl.BlockSpec((K, bk), lambda j: (0, j)),
            pl.BlockSpec((1, bk), lambda j: (0, j)),
            pl.BlockSpec((1, bk), lambda j: (0, j)),
            pl.BlockSpec((1, bk), lambda j: (0, j)),
        ],
        out_specs=pl.BlockSpec((M, bk), lambda j: (0, j)),
        out_shape=jax.ShapeDtypeStruct((M, Ko), jnp.float32),
    )(x, w, b.reshape(1, Ko), g.reshape(1, Ko), be.reshape(1, Ko))


def _mm_body(xr, wr, br, o):
    o[...] = jnp.dot(xr[...], wr[...], preferred_element_type=jnp.float32) + br[...]


def _mm(x, w, b, bm=1024):
    M, K = x.shape
    Ko = w.shape[1]
    return pl.pallas_call(
        _mm_body,
        grid=(M // bm,),
        in_specs=[
            pl.BlockSpec((bm, K), lambda i: (i, 0)),
            pl.BlockSpec((K, Ko), lambda i: (0, 0)),
            pl.BlockSpec((1, Ko), lambda i: (0, 0)),
        ],
        out_specs=pl.BlockSpec((bm, Ko), lambda i: (i, 0)),
        out_shape=jax.ShapeDtypeStruct((M, Ko), jnp.float32),
    )(x, w, b.reshape(1, Ko))


def kernel(x1, edge_index1, x1_batch, x2, edge_index2, x2_batch, ln0_w, ln0_b,
           gat_Wl, gat_bl, gat_Wr, gat_br, gat_att, gat_bias, sag_Wrel, sag_brel,
           sag_Wroot, nn_w, nn_b, ca_wq, ca_wk, ca_bias, ca_a, mlp_Wh, mlp_bh,
           mlp_g, mlp_be, mlp_Wo, mlp_bo):
    x = jnp.concatenate([x1, x2], axis=0)
    batch = jnp.stack([x1_batch, x2_batch]).astype(jnp.int32)
    srcg = jnp.stack([edge_index1[0], edge_index2[0] + N]).astype(jnp.int32)
    dstg = jnp.stack([edge_index1[1], edge_index2[1] + N]).astype(jnp.int32)
    srcl = jnp.stack([edge_index1[0], edge_index2[0]]).astype(jnp.int32)
    dstl = jnp.stack([edge_index1[1], edge_index2[1]]).astype(jnp.int32)

    zrow = jnp.zeros((D,), jnp.float32)
    wb0 = jnp.stack([ln0_w, ln0_b] + [zrow] * 6)
    x = _sc_ln0(x, batch, wb0)

    embs = []
    for i in range(NB):
        wlr = jnp.concatenate([gat_Wl[i], gat_Wr[i]], axis=1)
        blr = jnp.concatenate([gat_bl[i], gat_br[i]], axis=0)
        xl, xr = _mm2(x, wlr, blr)
        attp = jnp.stack([jnp.pad(gat_att[i][0], (0, D - C)),
                          jnp.pad(gat_att[i][1], (0, D - C))] + [zrow] * 6)
        xgh, zz = _sc_edge(xl, xr, srcg, dstg, dstl, attp)
        par = jnp.stack([
            sag_Wrel[i][:, 0],
            sag_Wroot[i][:, 0],
            gat_bias[i],
            nn_w[i],
            nn_b[i],
            jnp.full((D,), sag_brel[i][0], jnp.float32),
            zrow,
            zrow,
        ])
        x, emb = _sc_readout(xgh, zz, srcl, dstl, batch, par)
        embs.append(emb)

    embs = jnp.stack(embs, axis=0)
    h = _coattn(embs, ca_wq, ca_wk, ca_bias, ca_a)

    for l in range(3):
        h = _mlp_layer(h, mlp_Wh[l], mlp_bh[l], mlp_g[l], mlp_be[l])

    wo = jnp.concatenate([mlp_Wo, jnp.zeros((HID, 128 - OUT), jnp.float32)], axis=1)
    bo = jnp.concatenate([mlp_bo, jnp.zeros((128 - OUT,), jnp.float32)])
    out = _mm(h, wo, bo, bm=1024)
    return out[:, :OUT]


# speedup vs baseline: 15.3062x; 1.0983x over previous
import functools
import jax
import jax.numpy as jnp
from jax import lax
from jax.experimental import pallas as pl
from jax.experimental.pallas import tpu as pltpu
from jax.experimental.pallas import tpu_sc as plsc

N = 25600
E = 102400
B = 1024
D = 128
H = 2
C = 64
NB = 4
HID = 2048
OUT = 86

N2 = 2 * N
B2 = 2 * B

NPT = N // 16
EPT = E // 16
GPT = B // 16
CH = 128
NCH = N // CH
NIT = 13
ECH = EPT // CH

_MESH = plsc.VectorSubcoreMesh(core_axis_name="c", subcore_axis_name="s")


def _lane_masks():
    lanes = lax.iota(jnp.int32, 16)
    return [lanes == j for j in range(16)]


def _ins(acc, mask, scalar):
    return jnp.where(mask, jnp.full((16,), scalar, jnp.float32), acc)


def _hsum(v):
    parts = [v[j] for j in range(16)]
    while len(parts) > 1:
        parts = [parts[i] + parts[i + 1] for i in range(0, len(parts), 2)]
    return parts[0]


def _rsqrt_newton(x):
    i = lax.bitcast_convert_type(x, jnp.int32)
    i = jnp.int32(0x5F3759DF) - lax.shift_right_logical(i, jnp.int32(1))
    y = lax.bitcast_convert_type(i, jnp.float32)
    for _ in range(4):
        y = y * (1.5 - 0.5 * x * y * y)
    return y


def _zero2d(ref, rows, cols):
    z = jnp.zeros((16,), jnp.float32)

    def rbody(r, carry):
        for q in range(cols // 16):
            ref[r, pl.ds(16 * q, 16)] = z
        return carry

    lax.fori_loop(0, rows, rbody, 0)


def _fill1d(ref, n, val):
    v = jnp.full((16,), val, jnp.float32)

    def kbody(k, carry):
        ref[pl.ds(16 * k, 16)] = v
        return carry

    lax.fori_loop(0, n // 16, kbody, 0)


@functools.partial(
    pl.kernel,
    out_type=jax.ShapeDtypeStruct((N2, D), jnp.float32),
    mesh=_MESH,
    scratch_types=[
        pltpu.VMEM_SHARED((B,), jnp.float32),
        pltpu.VMEM_SHARED((B,), jnp.float32),
        pltpu.VMEM_SHARED((B,), jnp.float32),
        pltpu.VMEM((CH, D), jnp.float32),
        pltpu.VMEM((CH,), jnp.float32),
        pltpu.VMEM((CH,), jnp.float32),
        pltpu.VMEM((CH,), jnp.float32),
        pltpu.VMEM((CH,), jnp.float32),
        pltpu.VMEM((CH,), jnp.float32),
        pltpu.VMEM((CH,), jnp.int32),
        pltpu.VMEM((8, D), jnp.float32),
        pltpu.SemaphoreType.DMA,
    ],
)
def _sc_ln0(x_hbm, batch_hbm, wb_hbm, out_hbm, sum_sh, sq_sh, cnt_sh,
            xbuf, sumb, sqb, onesb, meanb, rstdb, bidx, pbuf, sem):
    c = lax.axis_index("c")
    s = lax.axis_index("s")
    masks = _lane_masks()
    _fill1d(sumb, CH, 0.0)
    _fill1d(onesb, CH, 1.0)
    pltpu.sync_copy(sumb.at[pl.ds(0, GPT)], sum_sh.at[pl.ds(s * GPT, GPT)])
    pltpu.sync_copy(sumb.at[pl.ds(0, GPT)], sq_sh.at[pl.ds(s * GPT, GPT)])
    pltpu.sync_copy(sumb.at[pl.ds(0, GPT)], cnt_sh.at[pl.ds(s * GPT, GPT)])
    pltpu.sync_copy(wb_hbm, pbuf)
    plsc.subcore_barrier()

    def p1(i, carry):
        chunk = s + 16 * i

        @pl.when(chunk < NCH)
        def _():
            base = chunk * CH
            gbase = c * N + base
            pltpu.sync_copy(x_hbm.at[pl.ds(gbase, CH)], xbuf)
            pltpu.sync_copy(batch_hbm.at[c, pl.ds(base, CH)], bidx)

            def grp(k, carry2):
                sumv = jnp.zeros((16,), jnp.float32)
                sqv = jnp.zeros((16,), jnp.float32)
                for j in range(16):
                    r = 16 * k + j
                    sv = jnp.zeros((16,), jnp.float32)
                    qv = jnp.zeros((16,), jnp.float32)
                    for q in range(D // 16):
                        v = xbuf[r, pl.ds(16 * q, 16)]
                        sv = sv + v
                        qv = qv + v * v
                    sumv = _ins(sumv, masks[j], _hsum(sv))
                    sqv = _ins(sqv, masks[j], _hsum(qv))
                sumb[pl.ds(16 * k, 16)] = sumv
                sqb[pl.ds(16 * k, 16)] = sqv
                return carry2

            lax.fori_loop(0, CH // 16, grp, 0)
            pltpu.sync_copy(sumb, sum_sh.at[bidx], add=True)
            pltpu.sync_copy(sqb, sq_sh.at[bidx], add=True)
            pltpu.sync_copy(onesb, cnt_sh.at[bidx], add=True)

        return carry

    lax.fori_loop(0, NIT, p1, 0)
    plsc.subcore_barrier()

    pltpu.sync_copy(sum_sh.at[pl.ds(s * GPT, GPT)], sumb.at[pl.ds(0, GPT)])
    pltpu.sync_copy(sq_sh.at[pl.ds(s * GPT, GPT)], sqb.at[pl.ds(0, GPT)])
    pltpu.sync_copy(cnt_sh.at[pl.ds(s * GPT, GPT)], onesb.at[pl.ds(0, GPT)])
    for g in range(GPT // 16):
        sl = pl.ds(16 * g, 16)
        cf = jnp.maximum(onesb[sl] * jnp.float32(D), 1.0)
        mean = sumb[sl] / cf
        var = jnp.maximum(sqb[sl] / cf - mean * mean, 0.0) + 1e-5
        sumb[sl] = mean
        sqb[sl] = _rsqrt_newton(var)
    pltpu.sync_copy(sumb.at[pl.ds(0, GPT)], sum_sh.at[pl.ds(s * GPT, GPT)])
    pltpu.sync_copy(sqb.at[pl.ds(0, GPT)], sq_sh.at[pl.ds(s * GPT, GPT)])
    plsc.subcore_barrier()

    def p3(i, carry):
        chunk = s + 16 * i

        @pl.when(chunk < NCH)
        def _():
            base = chunk * CH
            gbase = c * N + base
            pltpu.sync_copy(x_hbm.at[pl.ds(gbase, CH)], xbuf)
            pltpu.sync_copy(batch_hbm.at[c, pl.ds(base, CH)], bidx)
            pltpu.async_copy(sum_sh.at[bidx], meanb, sem).wait()
            pltpu.async_copy(sq_sh.at[bidx], rstdb, sem).wait()

            def rgrp(k, carry2):
                mv = meanb[pl.ds(16 * k, 16)]
                rv = rstdb[pl.ds(16 * k, 16)]
                for j in range(16):
                    r = 16 * k + j
                    mean = mv[j]
                    rstd = rv[j]
                    for q in range(D // 16):
                        v = xbuf[r, pl.ds(16 * q, 16)]
                        xbuf[r, pl.ds(16 * q, 16)] = (v - mean) * rstd * pbuf[0, pl.ds(16 * q, 16)] + pbuf[1, pl.ds(16 * q, 16)]
                return carry2

            lax.fori_loop(0, CH // 16, rgrp, 0)
            pltpu.sync_copy(xbuf, out_hbm.at[pl.ds(gbase, CH)])

        return carry

    lax.fori_loop(0, NIT, p3, 0)


NQ = 4
NH = N // NQ
NHP = NH + 8
HPT = NH // 16
ZPT = N // 16

@functools.partial(
    pl.kernel,
    out_type=[jax.ShapeDtypeStruct((N2, D), jnp.float32),
              jax.ShapeDtypeStruct((H, N2 // CH, CH), jnp.float32),
              jax.ShapeDtypeStruct((2 * E, D), jnp.float32)],
    mesh=_MESH,
    scratch_types=[
        pltpu.VMEM_SHARED((NHP, D), jnp.float32),
        pltpu.VMEM_SHARED((N,), jnp.float32),
        pltpu.VMEM_SHARED((N,), jnp.float32),
        pltpu.VMEM((80, D), jnp.float32),
        pltpu.VMEM((80,), jnp.float32),
        pltpu.VMEM((CH, D), jnp.float32),
        pltpu.VMEM((CH, D), jnp.float32),
        pltpu.VMEM((CH,), jnp.float32),
        pltpu.VMEM((CH,), jnp.float32),
        pltpu.VMEM((CH,), jnp.int32),
        pltpu.VMEM((CH,), jnp.int32),
        pltpu.VMEM((CH,), jnp.int32),
        pltpu.VMEM((8, D), jnp.float32),
        pltpu.SemaphoreType.DMA,
    ],
)
def _sc_edge(xl_hbm, xr_hbm, srcg_hbm, dstg_hbm, dstl_hbm,
             att_hbm, out_hbm, z_hbm, ebuf_hbm, num_sh, z0_sh, z1_sh,
             zbuf, zer1, gl, gr, ez0, ez1, isg, idg, idl, attv, sem):
    c = lax.axis_index("c")
    s = lax.axis_index("s")
    masks = _lane_masks()
    _zero2d(zbuf, 80, D)
    _fill1d(zer1, 80, 0.0)
    pltpu.sync_copy(att_hbm, attv)

    zb = s * ZPT
    for k in range(ZPT // 80):
        pltpu.sync_copy(zer1, z0_sh.at[pl.ds(zb + k * 80, 80)])
        pltpu.sync_copy(zer1, z1_sh.at[pl.ds(zb + k * 80, 80)])

    for p in range(NQ):
        rb = s * HPT
        for k in range(HPT // 80):
            pltpu.sync_copy(zbuf, num_sh.at[pl.ds(rb + k * 80, 80)])

        @pl.when(s == 15)
        def _():
            pltpu.sync_copy(zbuf.at[pl.ds(0, 8)], num_sh.at[pl.ds(NH, 8)])

        plsc.subcore_barrier()

        def edge_chunk(it, carry):
            base = s * EPT + it * CH
            pltpu.sync_copy(dstl_hbm.at[c, pl.ds(base, CH)], idl)

            if p == 0:
                pltpu.sync_copy(srcg_hbm.at[c, pl.ds(base, CH)], isg)
                pltpu.sync_copy(dstg_hbm.at[c, pl.ds(base, CH)], idg)
                pltpu.async_copy(xl_hbm.at[isg], gl, sem).wait()
                pltpu.async_copy(xr_hbm.at[idg], gr, sem).wait()

                def grp(k, carry2):
                    lv0 = jnp.zeros((16,), jnp.float32)
                    lv1 = jnp.zeros((16,), jnp.float32)
                    for j in range(16):
                        e = 16 * k + j
                        acc0 = jnp.zeros((16,), jnp.float32)
                        acc1 = jnp.zeros((16,), jnp.float32)
                        for q in range(4):
                            a = gl[e, pl.ds(16 * q, 16)]
                            b = gr[e, pl.ds(16 * q, 16)]
                            t = a + b
                            tl = jnp.maximum(t, 0.2 * t)
                            acc0 = acc0 + tl * attv[0, pl.ds(16 * q, 16)]
                            a1 = gl[e, pl.ds(64 + 16 * q, 16)]
                            b1 = gr[e, pl.ds(64 + 16 * q, 16)]
                            t1 = a1 + b1
                            tl1 = jnp.maximum(t1, 0.2 * t1)
                            acc1 = acc1 + tl1 * attv[1, pl.ds(16 * q, 16)]
                        lv0 = _ins(lv0, masks[j], _hsum(acc0))
                        lv1 = _ins(lv1, masks[j], _hsum(acc1))
                    ez0[pl.ds(16 * k, 16)] = jnp.exp(lv0)
                    ez1[pl.ds(16 * k, 16)] = jnp.exp(lv1)
                    return carry2

                lax.fori_loop(0, CH // 16, grp, 0)

                def wgt(k, carry2):
                    sv0 = ez0[pl.ds(16 * k, 16)]
                    sv1 = ez1[pl.ds(16 * k, 16)]
                    for j in range(16):
                        e = 16 * k + j
                        sc0 = sv0[j]
                        sc1 = sv1[j]
                        for q in range(4):
                            gl[e, pl.ds(16 * q, 16)] = gl[e, pl.ds(16 * q, 16)] * sc0
                            gl[e, pl.ds(64 + 16 * q, 16)] = gl[e, pl.ds(64 + 16 * q, 16)] * sc1
                    return carry2

                lax.fori_loop(0, CH // 16, wgt, 0)
                pltpu.sync_copy(ez0, z0_sh.at[idl], add=True)
                pltpu.sync_copy(ez1, z1_sh.at[idl], add=True)
                pltpu.sync_copy(gl, ebuf_hbm.at[pl.ds(c * E + base, CH)])
            else:
                pltpu.sync_copy(ebuf_hbm.at[pl.ds(c * E + base, CH)], gl)

            def idxk(k, carry2):
                sl = pl.ds(16 * k, 16)
                loc = idl[sl] - jnp.int32(p * NH)
                ok = (loc >= 0) & (loc < NH)
                idl[sl] = jnp.where(ok, loc, jnp.int32(NH))
                return carry2

            lax.fori_loop(0, CH // 16, idxk, 0)
            pltpu.sync_copy(gl, num_sh.at[idl], add=True)
            return carry

        lax.fori_loop(0, ECH, edge_chunk, 0)
        plsc.subcore_barrier()

        pltpu.sync_copy(num_sh.at[pl.ds(s * HPT, HPT)],
                        out_hbm.at[pl.ds(c * N + p * NH + s * HPT, HPT)])

        if p == 0:
            def zflush(i, carry):
                chunk = s + 16 * i

                @pl.when(chunk < NCH)
                def _():
                    gchunk = c * NCH + chunk
                    pltpu.sync_copy(z0_sh.at[pl.ds(chunk * CH, CH)], z_hbm.at[0, gchunk])
                    pltpu.sync_copy(z1_sh.at[pl.ds(chunk * CH, CH)], z_hbm.at[1, gchunk])

                return carry

            lax.fori_loop(0, NIT, zflush, 0)
        if p < NQ - 1:
            plsc.subcore_barrier()


@functools.partial(
    pl.kernel,
    out_type=[jax.ShapeDtypeStruct((N2, D), jnp.float32),
              jax.ShapeDtypeStruct((B2, D), jnp.float32)],
    mesh=_MESH,
    scratch_types=[
        pltpu.VMEM_SHARED((N,), jnp.float32),
        pltpu.VMEM_SHARED((N,), jnp.float32),
        pltpu.VMEM_SHARED((B,), jnp.float32),
        pltpu.VMEM_SHARED((B,), jnp.float32),
        pltpu.VMEM_SHARED((B,), jnp.float32),
        pltpu.VMEM_SHARED((B,), jnp.float32),
        pltpu.VMEM_SHARED((B, D), jnp.float32),
        pltpu.VMEM((CH, D), jnp.float32),
        pltpu.VMEM((CH, D), jnp.float32),
        pltpu.VMEM((8, D), jnp.float32),
        pltpu.VMEM((CH,), jnp.float32),
        pltpu.VMEM((CH,), jnp.float32),
        pltpu.VMEM((CH,), jnp.float32),
        pltpu.VMEM((CH,), jnp.float32),
        pltpu.VMEM((CH,), jnp.float32),
        pltpu.VMEM((CH,), jnp.float32),
        pltpu.VMEM((CH,), jnp.float32),
        pltpu.VMEM((CH,), jnp.float32),
        pltpu.VMEM((CH,), jnp.float32),
        pltpu.VMEM((CH,), jnp.int32),
        pltpu.VMEM((CH,), jnp.int32),
        pltpu.VMEM((CH,), jnp.int32),
        pltpu.VMEM((CH,), jnp.float32),
        pltpu.SemaphoreType.DMA,
    ],
)
def _sc_readout(xgh_hbm, z_hbm, srcl_hbm, dstl_hbm, batch_hbm, par_hbm,
                xnext_hbm, emb_hbm,
                xw_sh, sc_sh, zb_sh, sum_sh, sq_sh, cnt_sh, emb_sh,
                xbuf, wbuf, pbuf, sumb, sqb, onesb, xwbuf, scbuf, sbuf,
                zbv, z0ch, z1ch, bidx, isrc, idst, ebuf, sem):
    c = lax.axis_index("c")
    s = lax.axis_index("s")
    masks = _lane_masks()
    _zero2d(wbuf, CH, D)
    _fill1d(sumb, CH, 0.0)
    _fill1d(onesb, CH, 1.0)
    pltpu.sync_copy(sumb.at[pl.ds(0, GPT)], sum_sh.at[pl.ds(s * GPT, GPT)])
    pltpu.sync_copy(sumb.at[pl.ds(0, GPT)], sq_sh.at[pl.ds(s * GPT, GPT)])
    pltpu.sync_copy(sumb.at[pl.ds(0, GPT)], cnt_sh.at[pl.ds(s * GPT, GPT)])
    pltpu.sync_copy(sumb.at[pl.ds(0, GPT)], zb_sh.at[pl.ds(s * GPT, GPT)])
    pltpu.sync_copy(wbuf.at[pl.ds(0, GPT)], emb_sh.at[pl.ds(s * GPT, GPT)])
    pltpu.sync_copy(par_hbm, pbuf)
    plsc.subcore_barrier()

    def p1(i, carry):
        chunk = s + 16 * i

        @pl.when(chunk < NCH)
        def _():
            base = chunk * CH
            gbase = c * N + base
            pltpu.sync_copy(xgh_hbm.at[pl.ds(gbase, CH)], xbuf)
            pltpu.sync_copy(batch_hbm.at[c, pl.ds(base, CH)], bidx)
            pltpu.sync_copy(z_hbm.at[0, c * NCH + chunk], z0ch)
            pltpu.sync_copy(z_hbm.at[1, c * NCH + chunk], z1ch)
            brel = pbuf[5, pl.ds(0, 16)][0]

            def grp(k, carry2):
                sl = pl.ds(16 * k, 16)
                zv0 = 1.0 / (z0ch[sl] + 1e-16)
                zv1 = 1.0 / (z1ch[sl] + 1e-16)
                sumv = jnp.zeros((16,), jnp.float32)
                sqv = jnp.zeros((16,), jnp.float32)
                wrv = jnp.zeros((16,), jnp.float32)
                wtv = jnp.zeros((16,), jnp.float32)
                for j in range(16):
                    r = 16 * k + j
                    zi0 = zv0[j]
                    zi1 = zv1[j]
                    sv = jnp.zeros((16,), jnp.float32)
                    qv = jnp.zeros((16,), jnp.float32)
                    wr = jnp.zeros((16,), jnp.float32)
                    wt = jnp.zeros((16,), jnp.float32)
                    for q in range(D // 16):
                        zi = zi0 if q < 4 else zi1
                        v = xbuf[r, pl.ds(16 * q, 16)] * zi + pbuf[2, pl.ds(16 * q, 16)]
                        sv = sv + v
                        qv = qv + v * v
                        wr = wr + v * pbuf[0, pl.ds(16 * q, 16)]
                        wt = wt + v * pbuf[1, pl.ds(16 * q, 16)]
                    sumv = _ins(sumv, masks[j], _hsum(sv))
                    sqv = _ins(sqv, masks[j], _hsum(qv))
                    wrv = _ins(wrv, masks[j], _hsum(wr))
                    wtv = _ins(wtv, masks[j], _hsum(wt))
                sumb[pl.ds(16 * k, 16)] = sumv
                sqb[pl.ds(16 * k, 16)] = sqv
                xwbuf[pl.ds(16 * k, 16)] = wrv
                scbuf[pl.ds(16 * k, 16)] = wtv + brel
                return carry2

            lax.fori_loop(0, CH // 16, grp, 0)
            pltpu.sync_copy(xwbuf, xw_sh.at[pl.ds(base, CH)])
            pltpu.sync_copy(scbuf, sc_sh.at[pl.ds(base, CH)])
            pltpu.sync_copy(sumb, sum_sh.at[bidx], add=True)
            pltpu.sync_copy(sqb, sq_sh.at[bidx], add=True)
            pltpu.sync_copy(onesb, cnt_sh.at[bidx], add=True)

        return carry

    lax.fori_loop(0, NIT, p1, 0)
    plsc.subcore_barrier()

    def p2(it, carry):
        base = s * EPT + it * CH
        pltpu.sync_copy(srcl_hbm.at[c, pl.ds(base, CH)], isrc)
        pltpu.sync_copy(dstl_hbm.at[c, pl.ds(base, CH)], idst)
        pltpu.async_copy(xw_sh.at[isrc], ebuf, sem).wait()
        pltpu.sync_copy(ebuf, sc_sh.at[idst], add=True)
        return carry

    lax.fori_loop(0, ECH, p2, 0)
    plsc.subcore_barrier()

    def p3(i, carry):
        chunk = s + 16 * i

        @pl.when(chunk < NCH)
        def _():
            base = chunk * CH
            pltpu.sync_copy(sc_sh.at[pl.ds(base, CH)], sbuf)

            def expk(k, carry2):
                sbuf[pl.ds(16 * k, 16)] = jnp.exp(sbuf[pl.ds(16 * k, 16)])
                return carry2

            lax.fori_loop(0, CH // 16, expk, 0)
            pltpu.sync_copy(sbuf, sc_sh.at[pl.ds(base, CH)])
            pltpu.sync_copy(batch_hbm.at[c, pl.ds(base, CH)], bidx)
            pltpu.sync_copy(sbuf, zb_sh.at[bidx], add=True)

        return carry

    lax.fori_loop(0, NIT, p3, 0)
    plsc.subcore_barrier()

    pltpu.sync_copy(sum_sh.at[pl.ds(s * GPT, GPT)], sumb.at[pl.ds(0, GPT)])
    pltpu.sync_copy(sq_sh.at[pl.ds(s * GPT, GPT)], sqb.at[pl.ds(0, GPT)])
    pltpu.sync_copy(cnt_sh.at[pl.ds(s * GPT, GPT)], onesb.at[pl.ds(0, GPT)])
    for g in range(GPT // 16):
        sl = pl.ds(16 * g, 16)
        cf = jnp.maximum(onesb[sl] * jnp.float32(D), 1.0)
        mean = sumb[sl] / cf
        var = jnp.maximum(sqb[sl] / cf - mean * mean, 0.0) + 1e-5
        sumb[sl] = mean
        sqb[sl] = _rsqrt_newton(var)
    pltpu.sync_copy(sumb.at[pl.ds(0, GPT)], sum_sh.at[pl.ds(s * GPT, GPT)])
    pltpu.sync_copy(sqb.at[pl.ds(0, GPT)], sq_sh.at[pl.ds(s * GPT, GPT)])
    plsc.subcore_barrier()

    def p4(i, carry):
        chunk = s + 16 * i

        @pl.when(chunk < NCH)
        def _():
            base = chunk * CH
            gbase = c * N + base
            pltpu.sync_copy(xgh_hbm.at[pl.ds(gbase, CH)], xbuf)
            pltpu.sync_copy(batch_hbm.at[c, pl.ds(base, CH)], bidx)
            pltpu.sync_copy(sc_sh.at[pl.ds(base, CH)], sbuf)
            pltpu.sync_copy(z_hbm.at[0, c * NCH + chunk], z0ch)
            pltpu.sync_copy(z_hbm.at[1, c * NCH + chunk], z1ch)
            pltpu.async_copy(zb_sh.at[bidx], zbv, sem).wait()
            pltpu.async_copy(sum_sh.at[bidx], xwbuf, sem).wait()
            pltpu.async_copy(sq_sh.at[bidx], scbuf, sem).wait()

            def rgrp(k, carry2):
                sl = pl.ds(16 * k, 16)
                srv = sbuf[sl] / (zbv[sl] + 1e-16)
                zv0 = 1.0 / (z0ch[sl] + 1e-16)
                zv1 = 1.0 / (z1ch[sl] + 1e-16)
                mv = xwbuf[sl]
                rv = scbuf[sl]
                for j in range(16):
                    r = 16 * k + j
                    sr = srv[j]
                    zi0 = zv0[j]
                    zi1 = zv1[j]
                    mean = mv[j]
                    rstd = rv[j]
                    for q in range(D // 16):
                        zi = zi0 if q < 4 else zi1
                        v = xbuf[r, pl.ds(16 * q, 16)] * zi + pbuf[2, pl.ds(16 * q, 16)]
                        wbuf[r, pl.ds(16 * q, 16)] = v * sr
                        ln = (v - mean) * rstd * pbuf[3, pl.ds(16 * q, 16)] + pbuf[4, pl.ds(16 * q, 16)]
                        xbuf[r, pl.ds(16 * q, 16)] = jnp.where(ln > 0, ln, jnp.exp(ln) - 1.0)
                return carry2

            lax.fori_loop(0, CH // 16, rgrp, 0)
            pltpu.sync_copy(wbuf, emb_sh.at[bidx], add=True)
            pltpu.sync_copy(xbuf, xnext_hbm.at[pl.ds(gbase, CH)])

        return carry

    lax.fori_loop(0, NIT, p4, 0)
    plsc.subcore_barrier()
    pltpu.sync_copy(emb_sh.at[pl.ds(s * GPT, GPT)],
                    emb_hbm.at[pl.ds(c * B + s * GPT, GPT)])


def _mm2_body(xr, wr, br, o0, o1):
    y = jnp.dot(xr[...], wr[...], preferred_element_type=jnp.float32) + br[...]
    o0[...] = y[:, 0:128]
    o1[...] = y[:, 128:256]


def _mm2(x, w, b, bm=2048):
    M, K = x.shape
    outs = [jax.ShapeDtypeStruct((M, D), jnp.float32) for _ in range(2)]
    return pl.pallas_call(
        _mm2_body,
        grid=(M // bm,),
        in_specs=[
            pl.BlockSpec((bm, K), lambda i: (i, 0)),
            pl.BlockSpec((K, 2 * D), lambda i: (0, 0)),
            pl.BlockSpec((1, 2 * D), lambda i: (0, 0)),
        ],
        out_specs=[pl.BlockSpec((bm, D), lambda i: (i, 0)) for _ in range(2)],
        out_shape=outs,
    )(x, w, b.reshape(1, 2 * D))


def _coattn_body(e1, e2, wq, wk, cb, ca, o):
    r1 = [e1[i] for i in range(NB)]
    r2 = [e2[i] for i in range(NB)]
    keys = [jnp.dot(r, wk[...], preferred_element_type=jnp.float32) for r in r1]
    qrys = [jnp.dot(r, wq[...], preferred_element_type=jnp.float32) for r in r2]
    bias = cb[...]
    av = ca[...].reshape(D // 2, 1)
    r1n = [r / jnp.maximum(jnp.sqrt(jnp.sum(r * r, axis=1, keepdims=True)), 1e-12) for r in r1]
    r2n = [r / jnp.maximum(jnp.sqrt(jnp.sum(r * r, axis=1, keepdims=True)), 1e-12) for r in r2]
    for i in range(NB):
        for j in range(NB):
            att = jnp.dot(jnp.tanh(qrys[j] + keys[i] + bias), av,
                          preferred_element_type=jnp.float32)
            o[:, pl.ds(D * (NB * i + j), D)] = (r1n[i] + r2n[j]) * att


def _coattn(embs, wq, wk, cb, ca):
    return pl.pallas_call(
        _coattn_body,
        grid=(1,),
        in_specs=[
            pl.BlockSpec((NB, B, D), lambda i: (0, 0, 0)),
            pl.BlockSpec((NB, B, D), lambda i: (0, 1, 0)),
            pl.BlockSpec((D, D // 2), lambda i: (0, 0)),
            pl.BlockSpec((D, D // 2), lambda i: (0, 0)),
            pl.BlockSpec((1, D // 2), lambda i: (0, 0)),
            pl.BlockSpec((1, D // 2), lambda i: (0, 0)),
        ],
        out_specs=pl.BlockSpec((B, HID), lambda i: (0, 0)),
        out_shape=jax.ShapeDtypeStruct((B, HID), jnp.float32),
    )(embs, embs, wq, wk, cb.reshape(1, D // 2), ca.reshape(1, D // 2))


def _mlp_layer_body(xr, wr, br, gr, ber, o):
    h = jnp.dot(xr[...], wr[...], preferred_element_type=jnp.float32) + br[...]
    mu = jnp.mean(h, axis=0, keepdims=True)
    xc = h - mu
    var = jnp.mean(xc * xc, axis=0, keepdims=True)
    h = xc / jnp.sqrt(var + 1e-5) * gr[...] + ber[...]
    o[...] = jnp.maximum(h, 0.0)


def _mlp_layer(x, w, b, g, be, bk=512):
    M, K = x.shape
    Ko = w.shape[1]
    return pl.pallas_call(
        _mlp_layer_body,
        grid=(Ko // bk,),
        in_specs=[
            pl.BlockSpec((M, K), lambda j: (0, 0)),
            pl.BlockSpec((K, bk), lambda j: (0, j)),
            pl.BlockSpec((1, bk), lambda j: (0, j)),
            pl.BlockSpec((1, bk), lambda j: (0, j)),
            pl.BlockSpec((1, bk), lambda j: (0, j)),
        ],
        out_specs=pl.BlockSpec((M, bk), lambda j: (0, j)),
        out_shape=jax.ShapeDtypeStruct((M, Ko), jnp.float32),
    )(x, w, b.reshape(1, Ko), g.reshape(1, Ko), be.reshape(1, Ko))


def _mm_body(xr, wr, br, o):
    o[...] = jnp.dot(xr[...], wr[...], preferred_element_type=jnp.float32) + br[...]


def _mm(x, w, b, bm=1024):
    M, K = x.shape
    Ko = w.shape[1]
    return pl.pallas_call(
        _mm_body,
        grid=(M // bm,),
        in_specs=[
            pl.BlockSpec((bm, K), lambda i: (i, 0)),
            pl.BlockSpec((K, Ko), lambda i: (0, 0)),
            pl.BlockSpec((1, Ko), lambda i: (0, 0)),
        ],
        out_specs=pl.BlockSpec((bm, Ko), lambda i: (i, 0)),
        out_shape=jax.ShapeDtypeStruct((M, Ko), jnp.float32),
    )(x, w, b.reshape(1, Ko))


def kernel(x1, edge_index1, x1_batch, x2, edge_index2, x2_batch, ln0_w, ln0_b,
           gat_Wl, gat_bl, gat_Wr, gat_br, gat_att, gat_bias, sag_Wrel, sag_brel,
           sag_Wroot, nn_w, nn_b, ca_wq, ca_wk, ca_bias, ca_a, mlp_Wh, mlp_bh,
           mlp_g, mlp_be, mlp_Wo, mlp_bo):
    x = jnp.concatenate([x1, x2], axis=0)
    batch = jnp.stack([x1_batch, x2_batch]).astype(jnp.int32)
    srcg = jnp.stack([edge_index1[0], edge_index2[0] + N]).astype(jnp.int32)
    dstg = jnp.stack([edge_index1[1], edge_index2[1] + N]).astype(jnp.int32)
    srcl = jnp.stack([edge_index1[0], edge_index2[0]]).astype(jnp.int32)
    dstl = jnp.stack([edge_index1[1], edge_index2[1]]).astype(jnp.int32)

    zrow = jnp.zeros((D,), jnp.float32)
    wb0 = jnp.stack([ln0_w, ln0_b] + [zrow] * 6)
    x = _sc_ln0(x, batch, wb0)

    embs = []
    for i in range(NB):
        wlr = jnp.concatenate([gat_Wl[i], gat_Wr[i]], axis=1)
        blr = jnp.concatenate([gat_bl[i], gat_br[i]], axis=0)
        xl, xr = _mm2(x, wlr, blr)
        attp = jnp.stack([jnp.pad(gat_att[i][0], (0, D - C)),
                          jnp.pad(gat_att[i][1], (0, D - C))] + [zrow] * 6)
        xgh, zz, _ = _sc_edge(xl, xr, srcg, dstg, dstl, attp)
        par = jnp.stack([
            sag_Wrel[i][:, 0],
            sag_Wroot[i][:, 0],
            gat_bias[i],
            nn_w[i],
            nn_b[i],
            jnp.full((D,), sag_brel[i][0], jnp.float32),
            zrow,
            zrow,
        ])
        x, emb = _sc_readout(xgh, zz, srcl, dstl, batch, par)
        embs.append(emb)

    embs = jnp.stack(embs, axis=0)
    h = _coattn(embs, ca_wq, ca_wk, ca_bias, ca_a)

    for l in range(3):
        h = _mlp_layer(h, mlp_Wh[l], mlp_bh[l], mlp_g[l], mlp_be[l])

    wo = jnp.concatenate([mlp_Wo, jnp.zeros((HID, 128 - OUT), jnp.float32)], axis=1)
    bo = jnp.concatenate([mlp_bo, jnp.zeros((128 - OUT,), jnp.float32)])
    out = _mm(h, wo, bo, bm=1024)
    return out[:, :OUT]


# speedup vs baseline: 15.9969x; 1.0451x over previous
import functools
import jax
import jax.numpy as jnp
from jax import lax
from jax.experimental import pallas as pl
from jax.experimental.pallas import tpu as pltpu
from jax.experimental.pallas import tpu_sc as plsc

N = 25600
E = 102400
B = 1024
D = 128
H = 2
C = 64
NB = 4
HID = 2048
OUT = 86

N2 = 2 * N
B2 = 2 * B

NPT = N // 16
EPT = E // 16
GPT = B // 16
CH = 128
NCH = N // CH
NIT = 13
ECH = EPT // CH

_MESH = plsc.VectorSubcoreMesh(core_axis_name="c", subcore_axis_name="s")


def _lane_masks():
    lanes = lax.iota(jnp.int32, 16)
    return [lanes == j for j in range(16)]


def _ins(acc, mask, scalar):
    return jnp.where(mask, jnp.full((16,), scalar, jnp.float32), acc)


def _hsum(v):
    parts = [v[j] for j in range(16)]
    while len(parts) > 1:
        parts = [parts[i] + parts[i + 1] for i in range(0, len(parts), 2)]
    return parts[0]


def _rsqrt_newton(x):
    i = lax.bitcast_convert_type(x, jnp.int32)
    i = jnp.int32(0x5F3759DF) - lax.shift_right_logical(i, jnp.int32(1))
    y = lax.bitcast_convert_type(i, jnp.float32)
    for _ in range(4):
        y = y * (1.5 - 0.5 * x * y * y)
    return y


def _zero2d(ref, rows, cols):
    z = jnp.zeros((16,), jnp.float32)

    def rbody(r, carry):
        for q in range(cols // 16):
            ref[r, pl.ds(16 * q, 16)] = z
        return carry

    lax.fori_loop(0, rows, rbody, 0)


def _fill1d(ref, n, val):
    v = jnp.full((16,), val, jnp.float32)

    def kbody(k, carry):
        ref[pl.ds(16 * k, 16)] = v
        return carry

    lax.fori_loop(0, n // 16, kbody, 0)


@functools.partial(
    pl.kernel,
    out_type=jax.ShapeDtypeStruct((N2, D), jnp.float32),
    mesh=_MESH,
    scratch_types=[
        pltpu.VMEM_SHARED((B,), jnp.float32),
        pltpu.VMEM_SHARED((B,), jnp.float32),
        pltpu.VMEM_SHARED((B,), jnp.float32),
        pltpu.VMEM((CH, D), jnp.float32),
        pltpu.VMEM((CH,), jnp.float32),
        pltpu.VMEM((CH,), jnp.float32),
        pltpu.VMEM((CH,), jnp.float32),
        pltpu.VMEM((CH,), jnp.float32),
        pltpu.VMEM((CH,), jnp.float32),
        pltpu.VMEM((CH,), jnp.int32),
        pltpu.VMEM((8, D), jnp.float32),
        pltpu.SemaphoreType.DMA,
    ],
)
def _sc_ln0(x_hbm, batch_hbm, wb_hbm, out_hbm, sum_sh, sq_sh, cnt_sh,
            xbuf, sumb, sqb, onesb, meanb, rstdb, bidx, pbuf, sem):
    c = lax.axis_index("c")
    s = lax.axis_index("s")
    masks = _lane_masks()
    _fill1d(sumb, CH, 0.0)
    _fill1d(onesb, CH, 1.0)
    pltpu.sync_copy(sumb.at[pl.ds(0, GPT)], sum_sh.at[pl.ds(s * GPT, GPT)])
    pltpu.sync_copy(sumb.at[pl.ds(0, GPT)], sq_sh.at[pl.ds(s * GPT, GPT)])
    pltpu.sync_copy(sumb.at[pl.ds(0, GPT)], cnt_sh.at[pl.ds(s * GPT, GPT)])
    pltpu.sync_copy(wb_hbm, pbuf)
    plsc.subcore_barrier()

    def p1(i, carry):
        chunk = s + 16 * i

        @pl.when(chunk < NCH)
        def _():
            base = chunk * CH
            gbase = c * N + base
            pltpu.sync_copy(x_hbm.at[pl.ds(gbase, CH)], xbuf)
            pltpu.sync_copy(batch_hbm.at[c, pl.ds(base, CH)], bidx)

            def grp(k, carry2):
                sumv = jnp.zeros((16,), jnp.float32)
                sqv = jnp.zeros((16,), jnp.float32)
                for j in range(16):
                    r = 16 * k + j
                    sv = jnp.zeros((16,), jnp.float32)
                    qv = jnp.zeros((16,), jnp.float32)
                    for q in range(D // 16):
                        v = xbuf[r, pl.ds(16 * q, 16)]
                        sv = sv + v
                        qv = qv + v * v
                    sumv = _ins(sumv, masks[j], _hsum(sv))
                    sqv = _ins(sqv, masks[j], _hsum(qv))
                sumb[pl.ds(16 * k, 16)] = sumv
                sqb[pl.ds(16 * k, 16)] = sqv
                return carry2

            lax.fori_loop(0, CH // 16, grp, 0)
            pltpu.sync_copy(sumb, sum_sh.at[bidx], add=True)
            pltpu.sync_copy(sqb, sq_sh.at[bidx], add=True)
            pltpu.sync_copy(onesb, cnt_sh.at[bidx], add=True)

        return carry

    lax.fori_loop(0, NIT, p1, 0)
    plsc.subcore_barrier()

    pltpu.sync_copy(sum_sh.at[pl.ds(s * GPT, GPT)], sumb.at[pl.ds(0, GPT)])
    pltpu.sync_copy(sq_sh.at[pl.ds(s * GPT, GPT)], sqb.at[pl.ds(0, GPT)])
    pltpu.sync_copy(cnt_sh.at[pl.ds(s * GPT, GPT)], onesb.at[pl.ds(0, GPT)])
    for g in range(GPT // 16):
        sl = pl.ds(16 * g, 16)
        cf = jnp.maximum(onesb[sl] * jnp.float32(D), 1.0)
        mean = sumb[sl] / cf
        var = jnp.maximum(sqb[sl] / cf - mean * mean, 0.0) + 1e-5
        sumb[sl] = mean
        sqb[sl] = _rsqrt_newton(var)
    pltpu.sync_copy(sumb.at[pl.ds(0, GPT)], sum_sh.at[pl.ds(s * GPT, GPT)])
    pltpu.sync_copy(sqb.at[pl.ds(0, GPT)], sq_sh.at[pl.ds(s * GPT, GPT)])
    plsc.subcore_barrier()

    def p3(i, carry):
        chunk = s + 16 * i

        @pl.when(chunk < NCH)
        def _():
            base = chunk * CH
            gbase = c * N + base
            pltpu.sync_copy(x_hbm.at[pl.ds(gbase, CH)], xbuf)
            pltpu.sync_copy(batch_hbm.at[c, pl.ds(base, CH)], bidx)
            pltpu.async_copy(sum_sh.at[bidx], meanb, sem).wait()
            pltpu.async_copy(sq_sh.at[bidx], rstdb, sem).wait()

            def rgrp(k, carry2):
                mv = meanb[pl.ds(16 * k, 16)]
                rv = rstdb[pl.ds(16 * k, 16)]
                for j in range(16):
                    r = 16 * k + j
                    mean = mv[j]
                    rstd = rv[j]
                    for q in range(D // 16):
                        v = xbuf[r, pl.ds(16 * q, 16)]
                        xbuf[r, pl.ds(16 * q, 16)] = (v - mean) * rstd * pbuf[0, pl.ds(16 * q, 16)] + pbuf[1, pl.ds(16 * q, 16)]
                return carry2

            lax.fori_loop(0, CH // 16, rgrp, 0)
            pltpu.sync_copy(xbuf, out_hbm.at[pl.ds(gbase, CH)])

        return carry

    lax.fori_loop(0, NIT, p3, 0)


NQ = 4
NH = N // NQ
NHP = NH + 8
HPT = NH // 16
ZPT = N // 16

@functools.partial(
    pl.kernel,
    out_type=[jax.ShapeDtypeStruct((N2, D), jnp.float32),
              jax.ShapeDtypeStruct((H, N2 // CH, CH), jnp.float32),
              jax.ShapeDtypeStruct((2 * E, D), jnp.float32)],
    mesh=_MESH,
    scratch_types=[
        pltpu.VMEM_SHARED((NHP, D), jnp.float32),
        pltpu.VMEM_SHARED((N,), jnp.float32),
        pltpu.VMEM_SHARED((N,), jnp.float32),
        pltpu.VMEM((80, D), jnp.float32),
        pltpu.VMEM((80,), jnp.float32),
        pltpu.VMEM((CH, D), jnp.float32),
        pltpu.VMEM((CH, D), jnp.float32),
        pltpu.VMEM((CH,), jnp.float32),
        pltpu.VMEM((CH,), jnp.float32),
        pltpu.VMEM((CH,), jnp.int32),
        pltpu.VMEM((CH,), jnp.int32),
        pltpu.VMEM((CH,), jnp.int32),
        pltpu.VMEM((8, D), jnp.float32),
        pltpu.SemaphoreType.DMA,
        pltpu.SemaphoreType.DMA,
    ],
)
def _sc_edge(xl_hbm, xr_hbm, srcg_hbm, dstg_hbm, dstl_hbm,
             att_hbm, out_hbm, z_hbm, ebuf_hbm, num_sh, z0_sh, z1_sh,
             zbuf, zer1, gl, gr, ez0, ez1, isg, idg, idl, attv, sem, sem2):
    c = lax.axis_index("c")
    s = lax.axis_index("s")
    masks = _lane_masks()
    _zero2d(zbuf, 80, D)
    _fill1d(zer1, 80, 0.0)
    pltpu.sync_copy(att_hbm, attv)

    zb = s * ZPT
    for k in range(ZPT // 80):
        pltpu.sync_copy(zer1, z0_sh.at[pl.ds(zb + k * 80, 80)])
        pltpu.sync_copy(zer1, z1_sh.at[pl.ds(zb + k * 80, 80)])

    for p in range(NQ):
        rb = s * HPT
        for k in range(HPT // 80):
            pltpu.sync_copy(zbuf, num_sh.at[pl.ds(rb + k * 80, 80)])

        @pl.when(s == 15)
        def _():
            pltpu.sync_copy(zbuf.at[pl.ds(0, 8)], num_sh.at[pl.ds(NH, 8)])

        plsc.subcore_barrier()

        def _redirect(idxref):
            def idxk(k, carry2):
                sl = pl.ds(16 * k, 16)
                loc = idxref[sl] - jnp.int32(p * NH)
                ok = (loc >= 0) & (loc < NH)
                idxref[sl] = jnp.where(ok, loc, jnp.int32(NH))
                return carry2

            lax.fori_loop(0, CH // 16, idxk, 0)

        def edge_chunk(it, carry):
            base = s * EPT + it * CH
            pltpu.sync_copy(dstl_hbm.at[c, pl.ds(base, CH)], idl)

            if True:
                pltpu.sync_copy(srcg_hbm.at[c, pl.ds(base, CH)], isg)
                pltpu.sync_copy(dstg_hbm.at[c, pl.ds(base, CH)], idg)
                dl = pltpu.async_copy(xl_hbm.at[isg], gl, sem)
                dr = pltpu.async_copy(xr_hbm.at[idg], gr, sem2)
                dl.wait()
                dr.wait()

                def grp(k, carry2):
                    lv0 = jnp.zeros((16,), jnp.float32)
                    lv1 = jnp.zeros((16,), jnp.float32)
                    for j in range(16):
                        e = 16 * k + j
                        acc0 = jnp.zeros((16,), jnp.float32)
                        acc1 = jnp.zeros((16,), jnp.float32)
                        for q in range(4):
                            a = gl[e, pl.ds(16 * q, 16)]
                            b = gr[e, pl.ds(16 * q, 16)]
                            t = a + b
                            tl = jnp.maximum(t, 0.2 * t)
                            acc0 = acc0 + tl * attv[0, pl.ds(16 * q, 16)]
                            a1 = gl[e, pl.ds(64 + 16 * q, 16)]
                            b1 = gr[e, pl.ds(64 + 16 * q, 16)]
                            t1 = a1 + b1
                            tl1 = jnp.maximum(t1, 0.2 * t1)
                            acc1 = acc1 + tl1 * attv[1, pl.ds(16 * q, 16)]
                        lv0 = _ins(lv0, masks[j], _hsum(acc0))
                        lv1 = _ins(lv1, masks[j], _hsum(acc1))
                    ez0[pl.ds(16 * k, 16)] = jnp.exp(lv0)
                    ez1[pl.ds(16 * k, 16)] = jnp.exp(lv1)
                    return carry2

                lax.fori_loop(0, CH // 16, grp, 0)

                def wgt(k, carry2):
                    sv0 = ez0[pl.ds(16 * k, 16)]
                    sv1 = ez1[pl.ds(16 * k, 16)]
                    for j in range(16):
                        e = 16 * k + j
                        sc0 = sv0[j]
                        sc1 = sv1[j]
                        for q in range(4):
                            gl[e, pl.ds(16 * q, 16)] = gl[e, pl.ds(16 * q, 16)] * sc0
                            gl[e, pl.ds(64 + 16 * q, 16)] = gl[e, pl.ds(64 + 16 * q, 16)] * sc1
                    return carry2

                lax.fori_loop(0, CH // 16, wgt, 0)
                pltpu.sync_copy(ez0, z0_sh.at[idl], add=True)
                pltpu.sync_copy(ez1, z1_sh.at[idl], add=True)
                pltpu.sync_copy(gl, ebuf_hbm.at[pl.ds(c * E + base, CH)])

            _redirect(idl)
            pltpu.sync_copy(gl, num_sh.at[idl], add=True)
            return carry

        def edge_pair(i2, carry):
            base0 = s * EPT + (2 * i2) * CH
            base1 = base0 + CH
            d0 = pltpu.async_copy(ebuf_hbm.at[pl.ds(c * E + base0, CH)], gl, sem)
            d1 = pltpu.async_copy(ebuf_hbm.at[pl.ds(c * E + base1, CH)], gr, sem2)
            pltpu.sync_copy(dstl_hbm.at[c, pl.ds(base0, CH)], idl)
            pltpu.sync_copy(dstl_hbm.at[c, pl.ds(base1, CH)], idg)
            _redirect(idl)
            _redirect(idg)
            d0.wait()
            pltpu.sync_copy(gl, num_sh.at[idl], add=True)
            d1.wait()
            pltpu.sync_copy(gr, num_sh.at[idg], add=True)
            return carry

        if p == 0:
            lax.fori_loop(0, ECH, edge_chunk, 0)
        else:
            lax.fori_loop(0, ECH // 2, edge_pair, 0)
        plsc.subcore_barrier()

        pltpu.sync_copy(num_sh.at[pl.ds(s * HPT, HPT)],
                        out_hbm.at[pl.ds(c * N + p * NH + s * HPT, HPT)])

        if p == 0:
            def zflush(i, carry):
                chunk = s + 16 * i

                @pl.when(chunk < NCH)
                def _():
                    gchunk = c * NCH + chunk
                    pltpu.sync_copy(z0_sh.at[pl.ds(chunk * CH, CH)], z_hbm.at[0, gchunk])
                    pltpu.sync_copy(z1_sh.at[pl.ds(chunk * CH, CH)], z_hbm.at[1, gchunk])

                return carry

            lax.fori_loop(0, NIT, zflush, 0)
        if p < NQ - 1:
            plsc.subcore_barrier()


@functools.partial(
    pl.kernel,
    out_type=[jax.ShapeDtypeStruct((N2, D), jnp.float32),
              jax.ShapeDtypeStruct((B2, D), jnp.float32)],
    mesh=_MESH,
    scratch_types=[
        pltpu.VMEM_SHARED((N,), jnp.float32),
        pltpu.VMEM_SHARED((N,), jnp.float32),
        pltpu.VMEM_SHARED((B,), jnp.float32),
        pltpu.VMEM_SHARED((B,), jnp.float32),
        pltpu.VMEM_SHARED((B,), jnp.float32),
        pltpu.VMEM_SHARED((B,), jnp.float32),
        pltpu.VMEM_SHARED((B, D), jnp.float32),
        pltpu.VMEM((CH, D), jnp.float32),
        pltpu.VMEM((CH, D), jnp.float32),
        pltpu.VMEM((8, D), jnp.float32),
        pltpu.VMEM((CH,), jnp.float32),
        pltpu.VMEM((CH,), jnp.float32),
        pltpu.VMEM((CH,), jnp.float32),
        pltpu.VMEM((CH,), jnp.float32),
        pltpu.VMEM((CH,), jnp.float32),
        pltpu.VMEM((CH,), jnp.float32),
        pltpu.VMEM((CH,), jnp.float32),
        pltpu.VMEM((CH,), jnp.float32),
        pltpu.VMEM((CH,), jnp.float32),
        pltpu.VMEM((CH,), jnp.int32),
        pltpu.VMEM((CH,), jnp.int32),
        pltpu.VMEM((CH,), jnp.int32),
        pltpu.VMEM((CH,), jnp.float32),
        pltpu.SemaphoreType.DMA,
    ],
)
def _sc_readout(xgh_hbm, z_hbm, srcl_hbm, dstl_hbm, batch_hbm, par_hbm,
                xnext_hbm, emb_hbm,
                xw_sh, sc_sh, zb_sh, sum_sh, sq_sh, cnt_sh, emb_sh,
                xbuf, wbuf, pbuf, sumb, sqb, onesb, xwbuf, scbuf, sbuf,
                zbv, z0ch, z1ch, bidx, isrc, idst, ebuf, sem):
    c = lax.axis_index("c")
    s = lax.axis_index("s")
    masks = _lane_masks()
    _zero2d(wbuf, CH, D)
    _fill1d(sumb, CH, 0.0)
    _fill1d(onesb, CH, 1.0)
    pltpu.sync_copy(sumb.at[pl.ds(0, GPT)], sum_sh.at[pl.ds(s * GPT, GPT)])
    pltpu.sync_copy(sumb.at[pl.ds(0, GPT)], sq_sh.at[pl.ds(s * GPT, GPT)])
    pltpu.sync_copy(sumb.at[pl.ds(0, GPT)], cnt_sh.at[pl.ds(s * GPT, GPT)])
    pltpu.sync_copy(sumb.at[pl.ds(0, GPT)], zb_sh.at[pl.ds(s * GPT, GPT)])
    pltpu.sync_copy(wbuf.at[pl.ds(0, GPT)], emb_sh.at[pl.ds(s * GPT, GPT)])
    pltpu.sync_copy(par_hbm, pbuf)
    plsc.subcore_barrier()

    def p1(i, carry):
        chunk = s + 16 * i

        @pl.when(chunk < NCH)
        def _():
            base = chunk * CH
            gbase = c * N + base
            pltpu.sync_copy(xgh_hbm.at[pl.ds(gbase, CH)], xbuf)
            pltpu.sync_copy(batch_hbm.at[c, pl.ds(base, CH)], bidx)
            pltpu.sync_copy(z_hbm.at[0, c * NCH + chunk], z0ch)
            pltpu.sync_copy(z_hbm.at[1, c * NCH + chunk], z1ch)
            brel = pbuf[5, pl.ds(0, 16)][0]

            def grp(k, carry2):
                sl = pl.ds(16 * k, 16)
                zv0 = 1.0 / (z0ch[sl] + 1e-16)
                zv1 = 1.0 / (z1ch[sl] + 1e-16)
                sumv = jnp.zeros((16,), jnp.float32)
                sqv = jnp.zeros((16,), jnp.float32)
                wrv = jnp.zeros((16,), jnp.float32)
                wtv = jnp.zeros((16,), jnp.float32)
                for j in range(16):
                    r = 16 * k + j
                    zi0 = zv0[j]
                    zi1 = zv1[j]
                    sv = jnp.zeros((16,), jnp.float32)
                    qv = jnp.zeros((16,), jnp.float32)
                    wr = jnp.zeros((16,), jnp.float32)
                    wt = jnp.zeros((16,), jnp.float32)
                    for q in range(D // 16):
                        zi = zi0 if q < 4 else zi1
                        v = xbuf[r, pl.ds(16 * q, 16)] * zi + pbuf[2, pl.ds(16 * q, 16)]
                        sv = sv + v
                        qv = qv + v * v
                        wr = wr + v * pbuf[0, pl.ds(16 * q, 16)]
                        wt = wt + v * pbuf[1, pl.ds(16 * q, 16)]
                    sumv = _ins(sumv, masks[j], _hsum(sv))
                    sqv = _ins(sqv, masks[j], _hsum(qv))
                    wrv = _ins(wrv, masks[j], _hsum(wr))
                    wtv = _ins(wtv, masks[j], _hsum(wt))
                sumb[pl.ds(16 * k, 16)] = sumv
                sqb[pl.ds(16 * k, 16)] = sqv
                xwbuf[pl.ds(16 * k, 16)] = wrv
                scbuf[pl.ds(16 * k, 16)] = wtv + brel
                return carry2

            lax.fori_loop(0, CH // 16, grp, 0)
            pltpu.sync_copy(xwbuf, xw_sh.at[pl.ds(base, CH)])
            pltpu.sync_copy(scbuf, sc_sh.at[pl.ds(base, CH)])
            pltpu.sync_copy(sumb, sum_sh.at[bidx], add=True)
            pltpu.sync_copy(sqb, sq_sh.at[bidx], add=True)
            pltpu.sync_copy(onesb, cnt_sh.at[bidx], add=True)

        return carry

    lax.fori_loop(0, NIT, p1, 0)
    plsc.subcore_barrier()

    def p2(it, carry):
        base = s * EPT + it * CH
        pltpu.sync_copy(srcl_hbm.at[c, pl.ds(base, CH)], isrc)
        pltpu.sync_copy(dstl_hbm.at[c, pl.ds(base, CH)], idst)
        pltpu.async_copy(xw_sh.at[isrc], ebuf, sem).wait()
        pltpu.sync_copy(ebuf, sc_sh.at[idst], add=True)
        return carry

    lax.fori_loop(0, ECH, p2, 0)
    plsc.subcore_barrier()

    def p3(i, carry):
        chunk = s + 16 * i

        @pl.when(chunk < NCH)
        def _():
            base = chunk * CH
            pltpu.sync_copy(sc_sh.at[pl.ds(base, CH)], sbuf)

            def expk(k, carry2):
                sbuf[pl.ds(16 * k, 16)] = jnp.exp(sbuf[pl.ds(16 * k, 16)])
                return carry2

            lax.fori_loop(0, CH // 16, expk, 0)
            pltpu.sync_copy(sbuf, sc_sh.at[pl.ds(base, CH)])
            pltpu.sync_copy(batch_hbm.at[c, pl.ds(base, CH)], bidx)
            pltpu.sync_copy(sbuf, zb_sh.at[bidx], add=True)

        return carry

    lax.fori_loop(0, NIT, p3, 0)
    plsc.subcore_barrier()

    pltpu.sync_copy(sum_sh.at[pl.ds(s * GPT, GPT)], sumb.at[pl.ds(0, GPT)])
    pltpu.sync_copy(sq_sh.at[pl.ds(s * GPT, GPT)], sqb.at[pl.ds(0, GPT)])
    pltpu.sync_copy(cnt_sh.at[pl.ds(s * GPT, GPT)], onesb.at[pl.ds(0, GPT)])
    for g in range(GPT // 16):
        sl = pl.ds(16 * g, 16)
        cf = jnp.maximum(onesb[sl] * jnp.float32(D), 1.0)
        mean = sumb[sl] / cf
        var = jnp.maximum(sqb[sl] / cf - mean * mean, 0.0) + 1e-5
        sumb[sl] = mean
        sqb[sl] = _rsqrt_newton(var)
    pltpu.sync_copy(sumb.at[pl.ds(0, GPT)], sum_sh.at[pl.ds(s * GPT, GPT)])
    pltpu.sync_copy(sqb.at[pl.ds(0, GPT)], sq_sh.at[pl.ds(s * GPT, GPT)])
    plsc.subcore_barrier()

    def p4(i, carry):
        chunk = s + 16 * i

        @pl.when(chunk < NCH)
        def _():
            base = chunk * CH
            gbase = c * N + base
            pltpu.sync_copy(xgh_hbm.at[pl.ds(gbase, CH)], xbuf)
            pltpu.sync_copy(batch_hbm.at[c, pl.ds(base, CH)], bidx)
            pltpu.sync_copy(sc_sh.at[pl.ds(base, CH)], sbuf)
            pltpu.sync_copy(z_hbm.at[0, c * NCH + chunk], z0ch)
            pltpu.sync_copy(z_hbm.at[1, c * NCH + chunk], z1ch)
            pltpu.async_copy(zb_sh.at[bidx], zbv, sem).wait()
            pltpu.async_copy(sum_sh.at[bidx], xwbuf, sem).wait()
            pltpu.async_copy(sq_sh.at[bidx], scbuf, sem).wait()

            def rgrp(k, carry2):
                sl = pl.ds(16 * k, 16)
                srv = sbuf[sl] / (zbv[sl] + 1e-16)
                zv0 = 1.0 / (z0ch[sl] + 1e-16)
                zv1 = 1.0 / (z1ch[sl] + 1e-16)
                mv = xwbuf[sl]
                rv = scbuf[sl]
                for j in range(16):
                    r = 16 * k + j
                    sr = srv[j]
                    zi0 = zv0[j]
                    zi1 = zv1[j]
                    mean = mv[j]
                    rstd = rv[j]
                    for q in range(D // 16):
                        zi = zi0 if q < 4 else zi1
                        v = xbuf[r, pl.ds(16 * q, 16)] * zi + pbuf[2, pl.ds(16 * q, 16)]
                        wbuf[r, pl.ds(16 * q, 16)] = v * sr
                        ln = (v - mean) * rstd * pbuf[3, pl.ds(16 * q, 16)] + pbuf[4, pl.ds(16 * q, 16)]
                        xbuf[r, pl.ds(16 * q, 16)] = jnp.where(ln > 0, ln, jnp.exp(ln) - 1.0)
                return carry2

            lax.fori_loop(0, CH // 16, rgrp, 0)
            pltpu.sync_copy(wbuf, emb_sh.at[bidx], add=True)
            pltpu.sync_copy(xbuf, xnext_hbm.at[pl.ds(gbase, CH)])

        return carry

    lax.fori_loop(0, NIT, p4, 0)
    plsc.subcore_barrier()
    pltpu.sync_copy(emb_sh.at[pl.ds(s * GPT, GPT)],
                    emb_hbm.at[pl.ds(c * B + s * GPT, GPT)])


def _mm2_body(xr, wr, br, o0, o1):
    y = jnp.dot(xr[...], wr[...], preferred_element_type=jnp.float32) + br[...]
    o0[...] = y[:, 0:128]
    o1[...] = y[:, 128:256]


def _mm2(x, w, b, bm=2048):
    M, K = x.shape
    outs = [jax.ShapeDtypeStruct((M, D), jnp.float32) for _ in range(2)]
    return pl.pallas_call(
        _mm2_body,
        grid=(M // bm,),
        in_specs=[
            pl.BlockSpec((bm, K), lambda i: (i, 0)),
            pl.BlockSpec((K, 2 * D), lambda i: (0, 0)),
            pl.BlockSpec((1, 2 * D), lambda i: (0, 0)),
        ],
        out_specs=[pl.BlockSpec((bm, D), lambda i: (i, 0)) for _ in range(2)],
        out_shape=outs,
    )(x, w, b.reshape(1, 2 * D))


def _coattn_body(e1, e2, wq, wk, cb, ca, o):
    r1 = [e1[i] for i in range(NB)]
    r2 = [e2[i] for i in range(NB)]
    keys = [jnp.dot(r, wk[...], preferred_element_type=jnp.float32) for r in r1]
    qrys = [jnp.dot(r, wq[...], preferred_element_type=jnp.float32) for r in r2]
    bias = cb[...]
    av = ca[...].reshape(D // 2, 1)
    r1n = [r / jnp.maximum(jnp.sqrt(jnp.sum(r * r, axis=1, keepdims=True)), 1e-12) for r in r1]
    r2n = [r / jnp.maximum(jnp.sqrt(jnp.sum(r * r, axis=1, keepdims=True)), 1e-12) for r in r2]
    for i in range(NB):
        for j in range(NB):
            att = jnp.dot(jnp.tanh(qrys[j] + keys[i] + bias), av,
                          preferred_element_type=jnp.float32)
            o[:, pl.ds(D * (NB * i + j), D)] = (r1n[i] + r2n[j]) * att


def _coattn(embs, wq, wk, cb, ca):
    return pl.pallas_call(
        _coattn_body,
        grid=(1,),
        in_specs=[
            pl.BlockSpec((NB, B, D), lambda i: (0, 0, 0)),
            pl.BlockSpec((NB, B, D), lambda i: (0, 1, 0)),
            pl.BlockSpec((D, D // 2), lambda i: (0, 0)),
            pl.BlockSpec((D, D // 2), lambda i: (0, 0)),
            pl.BlockSpec((1, D // 2), lambda i: (0, 0)),
            pl.BlockSpec((1, D // 2), lambda i: (0, 0)),
        ],
        out_specs=pl.BlockSpec((B, HID), lambda i: (0, 0)),
        out_shape=jax.ShapeDtypeStruct((B, HID), jnp.float32),
    )(embs, embs, wq, wk, cb.reshape(1, D // 2), ca.reshape(1, D // 2))


def _mlp_layer_body(xr, wr, br, gr, ber, o):
    h = jnp.dot(xr[...], wr[...], preferred_element_type=jnp.float32) + br[...]
    mu = jnp.mean(h, axis=0, keepdims=True)
    xc = h - mu
    var = jnp.mean(xc * xc, axis=0, keepdims=True)
    h = xc / jnp.sqrt(var + 1e-5) * gr[...] + ber[...]
    o[...] = jnp.maximum(h, 0.0)


def _mlp_layer(x, w, b, g, be, bk=512):
    M, K = x.shape
    Ko = w.shape[1]
    return pl.pallas_call(
        _mlp_layer_body,
        grid=(Ko // bk,),
        in_specs=[
            pl.BlockSpec((M, K), lambda j: (0, 0)),
            pl.BlockSpec((K, bk), lambda j: (0, j)),
            pl.BlockSpec((1, bk), lambda j: (0, j)),
            pl.BlockSpec((1, bk), lambda j: (0, j)),
            pl.BlockSpec((1, bk), lambda j: (0, j)),
        ],
        out_specs=pl.BlockSpec((M, bk), lambda j: (0, j)),
        out_shape=jax.ShapeDtypeStruct((M, Ko), jnp.float32),
    )(x, w, b.reshape(1, Ko), g.reshape(1, Ko), be.reshape(1, Ko))


def _mm_body(xr, wr, br, o):
    o[...] = jnp.dot(xr[...], wr[...], preferred_element_type=jnp.float32) + br[...]


def _mm(x, w, b, bm=1024):
    M, K = x.shape
    Ko = w.shape[1]
    return pl.pallas_call(
        _mm_body,
        grid=(M // bm,),
        in_specs=[
            pl.BlockSpec((bm, K), lambda i: (i, 0)),
            pl.BlockSpec((K, Ko), lambda i: (0, 0)),
            pl.BlockSpec((1, Ko), lambda i: (0, 0)),
        ],
        out_specs=pl.BlockSpec((bm, Ko), lambda i: (i, 0)),
        out_shape=jax.ShapeDtypeStruct((M, Ko), jnp.float32),
    )(x, w, b.reshape(1, Ko))


def kernel(x1, edge_index1, x1_batch, x2, edge_index2, x2_batch, ln0_w, ln0_b,
           gat_Wl, gat_bl, gat_Wr, gat_br, gat_att, gat_bias, sag_Wrel, sag_brel,
           sag_Wroot, nn_w, nn_b, ca_wq, ca_wk, ca_bias, ca_a, mlp_Wh, mlp_bh,
           mlp_g, mlp_be, mlp_Wo, mlp_bo):
    x = jnp.concatenate([x1, x2], axis=0)
    batch = jnp.stack([x1_batch, x2_batch]).astype(jnp.int32)
    srcg = jnp.stack([edge_index1[0], edge_index2[0] + N]).astype(jnp.int32)
    dstg = jnp.stack([edge_index1[1], edge_index2[1] + N]).astype(jnp.int32)
    srcl = jnp.stack([edge_index1[0], edge_index2[0]]).astype(jnp.int32)
    dstl = jnp.stack([edge_index1[1], edge_index2[1]]).astype(jnp.int32)

    zrow = jnp.zeros((D,), jnp.float32)
    wb0 = jnp.stack([ln0_w, ln0_b] + [zrow] * 6)
    x = _sc_ln0(x, batch, wb0)

    embs = []
    for i in range(NB):
        wlr = jnp.concatenate([gat_Wl[i], gat_Wr[i]], axis=1)
        blr = jnp.concatenate([gat_bl[i], gat_br[i]], axis=0)
        xl, xr = _mm2(x, wlr, blr)
        attp = jnp.stack([jnp.pad(gat_att[i][0], (0, D - C)),
                          jnp.pad(gat_att[i][1], (0, D - C))] + [zrow] * 6)
        xgh, zz, _ = _sc_edge(xl, xr, srcg, dstg, dstl, attp)
        par = jnp.stack([
            sag_Wrel[i][:, 0],
            sag_Wroot[i][:, 0],
            gat_bias[i],
            nn_w[i],
            nn_b[i],
            jnp.full((D,), sag_brel[i][0], jnp.float32),
            zrow,
            zrow,
        ])
        x, emb = _sc_readout(xgh, zz, srcl, dstl, batch, par)
        embs.append(emb)

    embs = jnp.stack(embs, axis=0)
    h = _coattn(embs, ca_wq, ca_wk, ca_bias, ca_a)

    for l in range(3):
        h = _mlp_layer(h, mlp_Wh[l], mlp_bh[l], mlp_g[l], mlp_be[l])

    wo = jnp.concatenate([mlp_Wo, jnp.zeros((HID, 128 - OUT), jnp.float32)], axis=1)
    bo = jnp.concatenate([mlp_bo, jnp.zeros((128 - OUT,), jnp.float32)])
    out = _mm(h, wo, bo, bm=1024)
    return out[:, :OUT]


# speedup vs baseline: 16.1633x; 1.0104x over previous
import functools
import jax
import jax.numpy as jnp
from jax import lax
from jax.experimental import pallas as pl
from jax.experimental.pallas import tpu as pltpu
from jax.experimental.pallas import tpu_sc as plsc

N = 25600
E = 102400
B = 1024
D = 128
H = 2
C = 64
NB = 4
HID = 2048
OUT = 86

N2 = 2 * N
B2 = 2 * B

NPT = N // 16
EPT = E // 16
GPT = B // 16
CH = 128
NCH = N // CH
NIT = 13
ECH = EPT // CH

_MESH = plsc.VectorSubcoreMesh(core_axis_name="c", subcore_axis_name="s")


def _lane_masks():
    lanes = lax.iota(jnp.int32, 16)
    return [lanes == j for j in range(16)]


def _ins(acc, mask, scalar):
    return jnp.where(mask, jnp.full((16,), scalar, jnp.float32), acc)


def _hsum(v):
    parts = [v[j] for j in range(16)]
    while len(parts) > 1:
        parts = [parts[i] + parts[i + 1] for i in range(0, len(parts), 2)]
    return parts[0]


def _rsqrt_newton(x):
    i = lax.bitcast_convert_type(x, jnp.int32)
    i = jnp.int32(0x5F3759DF) - lax.shift_right_logical(i, jnp.int32(1))
    y = lax.bitcast_convert_type(i, jnp.float32)
    for _ in range(4):
        y = y * (1.5 - 0.5 * x * y * y)
    return y


def _zero2d(ref, rows, cols):
    z = jnp.zeros((16,), jnp.float32)

    def rbody(r, carry):
        for q in range(cols // 16):
            ref[r, pl.ds(16 * q, 16)] = z
        return carry

    lax.fori_loop(0, rows, rbody, 0)


def _fill1d(ref, n, val):
    v = jnp.full((16,), val, jnp.float32)

    def kbody(k, carry):
        ref[pl.ds(16 * k, 16)] = v
        return carry

    lax.fori_loop(0, n // 16, kbody, 0)


@functools.partial(
    pl.kernel,
    out_type=jax.ShapeDtypeStruct((N2, D), jnp.float32),
    mesh=_MESH,
    scratch_types=[
        pltpu.VMEM_SHARED((B,), jnp.float32),
        pltpu.VMEM_SHARED((B,), jnp.float32),
        pltpu.VMEM_SHARED((B,), jnp.float32),
        pltpu.VMEM((CH, D), jnp.float32),
        pltpu.VMEM((CH,), jnp.float32),
        pltpu.VMEM((CH,), jnp.float32),
        pltpu.VMEM((CH,), jnp.float32),
        pltpu.VMEM((CH,), jnp.float32),
        pltpu.VMEM((CH,), jnp.float32),
        pltpu.VMEM((CH,), jnp.int32),
        pltpu.VMEM((8, D), jnp.float32),
        pltpu.SemaphoreType.DMA,
    ],
)
def _sc_ln0(x_hbm, batch_hbm, wb_hbm, out_hbm, sum_sh, sq_sh, cnt_sh,
            xbuf, sumb, sqb, onesb, meanb, rstdb, bidx, pbuf, sem):
    c = lax.axis_index("c")
    s = lax.axis_index("s")
    masks = _lane_masks()
    _fill1d(sumb, CH, 0.0)
    _fill1d(onesb, CH, 1.0)
    pltpu.sync_copy(sumb.at[pl.ds(0, GPT)], sum_sh.at[pl.ds(s * GPT, GPT)])
    pltpu.sync_copy(sumb.at[pl.ds(0, GPT)], sq_sh.at[pl.ds(s * GPT, GPT)])
    pltpu.sync_copy(sumb.at[pl.ds(0, GPT)], cnt_sh.at[pl.ds(s * GPT, GPT)])
    pltpu.sync_copy(wb_hbm, pbuf)
    plsc.subcore_barrier()

    def p1(i, carry):
        chunk = s + 16 * i

        @pl.when(chunk < NCH)
        def _():
            base = chunk * CH
            gbase = c * N + base
            pltpu.sync_copy(x_hbm.at[pl.ds(gbase, CH)], xbuf)
            pltpu.sync_copy(batch_hbm.at[c, pl.ds(base, CH)], bidx)

            def grp(k, carry2):
                sumv = jnp.zeros((16,), jnp.float32)
                sqv = jnp.zeros((16,), jnp.float32)
                for j in range(16):
                    r = 16 * k + j
                    sv = jnp.zeros((16,), jnp.float32)
                    qv = jnp.zeros((16,), jnp.float32)
                    for q in range(D // 16):
                        v = xbuf[r, pl.ds(16 * q, 16)]
                        sv = sv + v
                        qv = qv + v * v
                    sumv = _ins(sumv, masks[j], _hsum(sv))
                    sqv = _ins(sqv, masks[j], _hsum(qv))
                sumb[pl.ds(16 * k, 16)] = sumv
                sqb[pl.ds(16 * k, 16)] = sqv
                return carry2

            lax.fori_loop(0, CH // 16, grp, 0)
            pltpu.sync_copy(sumb, sum_sh.at[bidx], add=True)
            pltpu.sync_copy(sqb, sq_sh.at[bidx], add=True)
            pltpu.sync_copy(onesb, cnt_sh.at[bidx], add=True)

        return carry

    lax.fori_loop(0, NIT, p1, 0)
    plsc.subcore_barrier()

    pltpu.sync_copy(sum_sh.at[pl.ds(s * GPT, GPT)], sumb.at[pl.ds(0, GPT)])
    pltpu.sync_copy(sq_sh.at[pl.ds(s * GPT, GPT)], sqb.at[pl.ds(0, GPT)])
    pltpu.sync_copy(cnt_sh.at[pl.ds(s * GPT, GPT)], onesb.at[pl.ds(0, GPT)])
    for g in range(GPT // 16):
        sl = pl.ds(16 * g, 16)
        cf = jnp.maximum(onesb[sl] * jnp.float32(D), 1.0)
        mean = sumb[sl] / cf
        var = jnp.maximum(sqb[sl] / cf - mean * mean, 0.0) + 1e-5
        sumb[sl] = mean
        sqb[sl] = _rsqrt_newton(var)
    pltpu.sync_copy(sumb.at[pl.ds(0, GPT)], sum_sh.at[pl.ds(s * GPT, GPT)])
    pltpu.sync_copy(sqb.at[pl.ds(0, GPT)], sq_sh.at[pl.ds(s * GPT, GPT)])
    plsc.subcore_barrier()

    def p3(i, carry):
        chunk = s + 16 * i

        @pl.when(chunk < NCH)
        def _():
            base = chunk * CH
            gbase = c * N + base
            pltpu.sync_copy(x_hbm.at[pl.ds(gbase, CH)], xbuf)
            pltpu.sync_copy(batch_hbm.at[c, pl.ds(base, CH)], bidx)
            pltpu.async_copy(sum_sh.at[bidx], meanb, sem).wait()
            pltpu.async_copy(sq_sh.at[bidx], rstdb, sem).wait()

            def rgrp(k, carry2):
                mv = meanb[pl.ds(16 * k, 16)]
                rv = rstdb[pl.ds(16 * k, 16)]
                for j in range(16):
                    r = 16 * k + j
                    mean = mv[j]
                    rstd = rv[j]
                    for q in range(D // 16):
                        v = xbuf[r, pl.ds(16 * q, 16)]
                        xbuf[r, pl.ds(16 * q, 16)] = (v - mean) * rstd * pbuf[0, pl.ds(16 * q, 16)] + pbuf[1, pl.ds(16 * q, 16)]
                return carry2

            lax.fori_loop(0, CH // 16, rgrp, 0)
            pltpu.sync_copy(xbuf, out_hbm.at[pl.ds(gbase, CH)])

        return carry

    lax.fori_loop(0, NIT, p3, 0)


NQ = 4
NH = N // NQ
NHP = NH + 8
HPT = NH // 16
ZPT = N // 16

@functools.partial(
    pl.kernel,
    out_type=[jax.ShapeDtypeStruct((N2, D), jnp.float32),
              jax.ShapeDtypeStruct((H, N2 // CH, CH), jnp.float32),
              jax.ShapeDtypeStruct((2 * E, D), jnp.float32)],
    mesh=_MESH,
    scratch_types=[
        pltpu.VMEM_SHARED((NHP, D), jnp.float32),
        pltpu.VMEM_SHARED((N,), jnp.float32),
        pltpu.VMEM_SHARED((N,), jnp.float32),
        pltpu.VMEM((80, D), jnp.float32),
        pltpu.VMEM((80,), jnp.float32),
        pltpu.VMEM((CH, D), jnp.float32),
        pltpu.VMEM((CH, D), jnp.float32),
        pltpu.VMEM((CH,), jnp.float32),
        pltpu.VMEM((CH,), jnp.float32),
        pltpu.VMEM((CH,), jnp.int32),
        pltpu.VMEM((CH,), jnp.int32),
        pltpu.VMEM((CH,), jnp.int32),
        pltpu.VMEM((8, D), jnp.float32),
        pltpu.SemaphoreType.DMA,
        pltpu.SemaphoreType.DMA,
    ],
)
def _sc_edge(xl_hbm, xr_hbm, srcg_hbm, dstg_hbm, dstl_hbm,
             att_hbm, out_hbm, z_hbm, ebuf_hbm, num_sh, z0_sh, z1_sh,
             zbuf, zer1, gl, gr, ez0, ez1, isg, idg, idl, attv, sem, sem2):
    c = lax.axis_index("c")
    s = lax.axis_index("s")
    masks = _lane_masks()
    _zero2d(zbuf, 80, D)
    _fill1d(zer1, 80, 0.0)
    pltpu.sync_copy(att_hbm, attv)

    zb = s * ZPT
    for k in range(ZPT // 80):
        pltpu.sync_copy(zer1, z0_sh.at[pl.ds(zb + k * 80, 80)])
        pltpu.sync_copy(zer1, z1_sh.at[pl.ds(zb + k * 80, 80)])

    for p in range(NQ):
        rb = s * HPT
        for k in range(HPT // 80):
            pltpu.sync_copy(zbuf, num_sh.at[pl.ds(rb + k * 80, 80)])

        @pl.when(s == 15)
        def _():
            pltpu.sync_copy(zbuf.at[pl.ds(0, 8)], num_sh.at[pl.ds(NH, 8)])

        plsc.subcore_barrier()

        def _redirect(idxref):
            def idxk(k, carry2):
                sl = pl.ds(16 * k, 16)
                loc = idxref[sl] - jnp.int32(p * NH)
                ok = (loc >= 0) & (loc < NH)
                idxref[sl] = jnp.where(ok, loc, jnp.int32(NH))
                return carry2

            lax.fori_loop(0, CH // 16, idxk, 0)

        def edge_chunk(it, carry):
            base = s * EPT + it * CH
            pltpu.sync_copy(dstl_hbm.at[c, pl.ds(base, CH)], idl)

            if True:
                pltpu.sync_copy(srcg_hbm.at[c, pl.ds(base, CH)], isg)
                pltpu.sync_copy(dstg_hbm.at[c, pl.ds(base, CH)], idg)
                dl = pltpu.async_copy(xl_hbm.at[isg], gl, sem)
                dr = pltpu.async_copy(xr_hbm.at[idg], gr, sem2)
                dl.wait()
                dr.wait()

                def grp(k, carry2):
                    lv0 = jnp.zeros((16,), jnp.float32)
                    lv1 = jnp.zeros((16,), jnp.float32)
                    for j in range(16):
                        e = 16 * k + j
                        acc0 = jnp.zeros((16,), jnp.float32)
                        acc1 = jnp.zeros((16,), jnp.float32)
                        for q in range(4):
                            a = gl[e, pl.ds(16 * q, 16)]
                            b = gr[e, pl.ds(16 * q, 16)]
                            t = a + b
                            tl = jnp.maximum(t, 0.2 * t)
                            acc0 = acc0 + tl * attv[0, pl.ds(16 * q, 16)]
                            a1 = gl[e, pl.ds(64 + 16 * q, 16)]
                            b1 = gr[e, pl.ds(64 + 16 * q, 16)]
                            t1 = a1 + b1
                            tl1 = jnp.maximum(t1, 0.2 * t1)
                            acc1 = acc1 + tl1 * attv[1, pl.ds(16 * q, 16)]
                        lv0 = _ins(lv0, masks[j], _hsum(acc0))
                        lv1 = _ins(lv1, masks[j], _hsum(acc1))
                    ez0[pl.ds(16 * k, 16)] = jnp.exp(lv0)
                    ez1[pl.ds(16 * k, 16)] = jnp.exp(lv1)
                    return carry2

                lax.fori_loop(0, CH // 16, grp, 0)

                def wgt(k, carry2):
                    sv0 = ez0[pl.ds(16 * k, 16)]
                    sv1 = ez1[pl.ds(16 * k, 16)]
                    for j in range(16):
                        e = 16 * k + j
                        sc0 = sv0[j]
                        sc1 = sv1[j]
                        for q in range(4):
                            gl[e, pl.ds(16 * q, 16)] = gl[e, pl.ds(16 * q, 16)] * sc0
                            gl[e, pl.ds(64 + 16 * q, 16)] = gl[e, pl.ds(64 + 16 * q, 16)] * sc1
                    return carry2

                lax.fori_loop(0, CH // 16, wgt, 0)
                pltpu.sync_copy(ez0, z0_sh.at[idl], add=True)
                pltpu.sync_copy(ez1, z1_sh.at[idl], add=True)
                pltpu.sync_copy(gl, ebuf_hbm.at[pl.ds(c * E + base, CH)])

            _redirect(idl)
            pltpu.sync_copy(gl, num_sh.at[idl], add=True)
            return carry

        def edge_pair(i2, carry):
            base0 = s * EPT + (2 * i2) * CH
            base1 = base0 + CH
            d0 = pltpu.async_copy(ebuf_hbm.at[pl.ds(c * E + base0, CH)], gl, sem)
            d1 = pltpu.async_copy(ebuf_hbm.at[pl.ds(c * E + base1, CH)], gr, sem2)
            pltpu.sync_copy(dstl_hbm.at[c, pl.ds(base0, CH)], idl)
            pltpu.sync_copy(dstl_hbm.at[c, pl.ds(base1, CH)], idg)
            _redirect(idl)
            _redirect(idg)
            d0.wait()
            pltpu.sync_copy(gl, num_sh.at[idl], add=True)
            d1.wait()
            pltpu.sync_copy(gr, num_sh.at[idg], add=True)
            return carry

        if p == 0:
            lax.fori_loop(0, ECH, edge_chunk, 0)
        else:
            lax.fori_loop(0, ECH // 2, edge_pair, 0)
        plsc.subcore_barrier()

        pltpu.sync_copy(num_sh.at[pl.ds(s * HPT, HPT)],
                        out_hbm.at[pl.ds(c * N + p * NH + s * HPT, HPT)])

        if p == 0:
            def zflush(i, carry):
                chunk = s + 16 * i

                @pl.when(chunk < NCH)
                def _():
                    gchunk = c * NCH + chunk
                    pltpu.sync_copy(z0_sh.at[pl.ds(chunk * CH, CH)], z_hbm.at[0, gchunk])
                    pltpu.sync_copy(z1_sh.at[pl.ds(chunk * CH, CH)], z_hbm.at[1, gchunk])

                return carry

            lax.fori_loop(0, NIT, zflush, 0)
        if p < NQ - 1:
            plsc.subcore_barrier()


@functools.partial(
    pl.kernel,
    out_type=[jax.ShapeDtypeStruct((N2, D), jnp.float32),
              jax.ShapeDtypeStruct((B2, D), jnp.float32)],
    mesh=_MESH,
    scratch_types=[
        pltpu.VMEM_SHARED((N,), jnp.float32),
        pltpu.VMEM_SHARED((N,), jnp.float32),
        pltpu.VMEM_SHARED((B,), jnp.float32),
        pltpu.VMEM_SHARED((B,), jnp.float32),
        pltpu.VMEM_SHARED((B,), jnp.float32),
        pltpu.VMEM_SHARED((B,), jnp.float32),
        pltpu.VMEM_SHARED((B, D), jnp.float32),
        pltpu.VMEM((CH, D), jnp.float32),
        pltpu.VMEM((CH, D), jnp.float32),
        pltpu.VMEM((8, D), jnp.float32),
        pltpu.VMEM((CH,), jnp.float32),
        pltpu.VMEM((CH,), jnp.float32),
        pltpu.VMEM((CH,), jnp.float32),
        pltpu.VMEM((CH,), jnp.float32),
        pltpu.VMEM((CH,), jnp.float32),
        pltpu.VMEM((CH,), jnp.float32),
        pltpu.VMEM((CH,), jnp.float32),
        pltpu.VMEM((CH,), jnp.float32),
        pltpu.VMEM((CH,), jnp.float32),
        pltpu.VMEM((CH,), jnp.int32),
        pltpu.VMEM((CH,), jnp.int32),
        pltpu.VMEM((CH,), jnp.int32),
        pltpu.VMEM((CH,), jnp.float32),
        pltpu.SemaphoreType.DMA,
        pltpu.SemaphoreType.DMA,
        pltpu.SemaphoreType.DMA,
        pltpu.SemaphoreType.DMA,
    ],
)
def _sc_readout(xgh_hbm, z_hbm, srcl_hbm, dstl_hbm, batch_hbm, par_hbm,
                xnext_hbm, emb_hbm,
                xw_sh, sc_sh, zb_sh, sum_sh, sq_sh, cnt_sh, emb_sh,
                xbuf, wbuf, pbuf, sumb, sqb, onesb, xwbuf, scbuf, sbuf,
                zbv, z0ch, z1ch, bidx, isrc, idst, ebuf, sem, sem2, sem3, sem4):
    c = lax.axis_index("c")
    s = lax.axis_index("s")
    masks = _lane_masks()
    _zero2d(wbuf, CH, D)
    _fill1d(sumb, CH, 0.0)
    _fill1d(onesb, CH, 1.0)
    pltpu.sync_copy(sumb.at[pl.ds(0, GPT)], sum_sh.at[pl.ds(s * GPT, GPT)])
    pltpu.sync_copy(sumb.at[pl.ds(0, GPT)], sq_sh.at[pl.ds(s * GPT, GPT)])
    pltpu.sync_copy(sumb.at[pl.ds(0, GPT)], cnt_sh.at[pl.ds(s * GPT, GPT)])
    pltpu.sync_copy(sumb.at[pl.ds(0, GPT)], zb_sh.at[pl.ds(s * GPT, GPT)])
    pltpu.sync_copy(wbuf.at[pl.ds(0, GPT)], emb_sh.at[pl.ds(s * GPT, GPT)])
    pltpu.sync_copy(par_hbm, pbuf)
    plsc.subcore_barrier()

    def p1(i, carry):
        chunk = s + 16 * i

        @pl.when(chunk < NCH)
        def _():
            base = chunk * CH
            gbase = c * N + base
            d1 = pltpu.async_copy(xgh_hbm.at[pl.ds(gbase, CH)], xbuf, sem)
            d2 = pltpu.async_copy(batch_hbm.at[c, pl.ds(base, CH)], bidx, sem2)
            d3 = pltpu.async_copy(z_hbm.at[0, c * NCH + chunk], z0ch, sem3)
            d4 = pltpu.async_copy(z_hbm.at[1, c * NCH + chunk], z1ch, sem4)
            d1.wait()
            d2.wait()
            d3.wait()
            d4.wait()
            brel = pbuf[5, pl.ds(0, 16)][0]

            def grp(k, carry2):
                sl = pl.ds(16 * k, 16)
                zv0 = 1.0 / (z0ch[sl] + 1e-16)
                zv1 = 1.0 / (z1ch[sl] + 1e-16)
                sumv = jnp.zeros((16,), jnp.float32)
                sqv = jnp.zeros((16,), jnp.float32)
                wrv = jnp.zeros((16,), jnp.float32)
                wtv = jnp.zeros((16,), jnp.float32)
                for j in range(16):
                    r = 16 * k + j
                    zi0 = zv0[j]
                    zi1 = zv1[j]
                    sv = jnp.zeros((16,), jnp.float32)
                    qv = jnp.zeros((16,), jnp.float32)
                    wr = jnp.zeros((16,), jnp.float32)
                    wt = jnp.zeros((16,), jnp.float32)
                    for q in range(D // 16):
                        zi = zi0 if q < 4 else zi1
                        v = xbuf[r, pl.ds(16 * q, 16)] * zi + pbuf[2, pl.ds(16 * q, 16)]
                        sv = sv + v
                        qv = qv + v * v
                        wr = wr + v * pbuf[0, pl.ds(16 * q, 16)]
                        wt = wt + v * pbuf[1, pl.ds(16 * q, 16)]
                    sumv = _ins(sumv, masks[j], _hsum(sv))
                    sqv = _ins(sqv, masks[j], _hsum(qv))
                    wrv = _ins(wrv, masks[j], _hsum(wr))
                    wtv = _ins(wtv, masks[j], _hsum(wt))
                sumb[pl.ds(16 * k, 16)] = sumv
                sqb[pl.ds(16 * k, 16)] = sqv
                xwbuf[pl.ds(16 * k, 16)] = wrv
                scbuf[pl.ds(16 * k, 16)] = wtv + brel
                return carry2

            lax.fori_loop(0, CH // 16, grp, 0)
            pltpu.sync_copy(xwbuf, xw_sh.at[pl.ds(base, CH)])
            pltpu.sync_copy(scbuf, sc_sh.at[pl.ds(base, CH)])
            pltpu.sync_copy(sumb, sum_sh.at[bidx], add=True)
            pltpu.sync_copy(sqb, sq_sh.at[bidx], add=True)
            pltpu.sync_copy(onesb, cnt_sh.at[bidx], add=True)

        return carry

    lax.fori_loop(0, NIT, p1, 0)
    plsc.subcore_barrier()

    def p2(it, carry):
        base = s * EPT + it * CH
        pltpu.sync_copy(srcl_hbm.at[c, pl.ds(base, CH)], isrc)
        pltpu.sync_copy(dstl_hbm.at[c, pl.ds(base, CH)], idst)
        pltpu.async_copy(xw_sh.at[isrc], ebuf, sem).wait()
        pltpu.sync_copy(ebuf, sc_sh.at[idst], add=True)
        return carry

    lax.fori_loop(0, ECH, p2, 0)
    plsc.subcore_barrier()

    def p3(i, carry):
        chunk = s + 16 * i

        @pl.when(chunk < NCH)
        def _():
            base = chunk * CH
            pltpu.sync_copy(sc_sh.at[pl.ds(base, CH)], sbuf)

            def expk(k, carry2):
                sbuf[pl.ds(16 * k, 16)] = jnp.exp(sbuf[pl.ds(16 * k, 16)])
                return carry2

            lax.fori_loop(0, CH // 16, expk, 0)
            pltpu.sync_copy(sbuf, sc_sh.at[pl.ds(base, CH)])
            pltpu.sync_copy(batch_hbm.at[c, pl.ds(base, CH)], bidx)
            pltpu.sync_copy(sbuf, zb_sh.at[bidx], add=True)

        return carry

    lax.fori_loop(0, NIT, p3, 0)
    plsc.subcore_barrier()

    pltpu.sync_copy(sum_sh.at[pl.ds(s * GPT, GPT)], sumb.at[pl.ds(0, GPT)])
    pltpu.sync_copy(sq_sh.at[pl.ds(s * GPT, GPT)], sqb.at[pl.ds(0, GPT)])
    pltpu.sync_copy(cnt_sh.at[pl.ds(s * GPT, GPT)], onesb.at[pl.ds(0, GPT)])
    for g in range(GPT // 16):
        sl = pl.ds(16 * g, 16)
        cf = jnp.maximum(onesb[sl] * jnp.float32(D), 1.0)
        mean = sumb[sl] / cf
        var = jnp.maximum(sqb[sl] / cf - mean * mean, 0.0) + 1e-5
        sumb[sl] = mean
        sqb[sl] = _rsqrt_newton(var)
    pltpu.sync_copy(sumb.at[pl.ds(0, GPT)], sum_sh.at[pl.ds(s * GPT, GPT)])
    pltpu.sync_copy(sqb.at[pl.ds(0, GPT)], sq_sh.at[pl.ds(s * GPT, GPT)])
    plsc.subcore_barrier()

    def p4(i, carry):
        chunk = s + 16 * i

        @pl.when(chunk < NCH)
        def _():
            base = chunk * CH
            gbase = c * N + base
            d1 = pltpu.async_copy(xgh_hbm.at[pl.ds(gbase, CH)], xbuf, sem)
            d2 = pltpu.async_copy(batch_hbm.at[c, pl.ds(base, CH)], bidx, sem2)
            d3 = pltpu.async_copy(z_hbm.at[0, c * NCH + chunk], z0ch, sem3)
            d4 = pltpu.async_copy(z_hbm.at[1, c * NCH + chunk], z1ch, sem4)
            pltpu.sync_copy(sc_sh.at[pl.ds(base, CH)], sbuf)
            d2.wait()
            d3.wait()
            d4.wait()
            d5 = pltpu.async_copy(zb_sh.at[bidx], zbv, sem2)
            d6 = pltpu.async_copy(sum_sh.at[bidx], xwbuf, sem3)
            d7 = pltpu.async_copy(sq_sh.at[bidx], scbuf, sem4)
            d1.wait()
            d5.wait()
            d6.wait()
            d7.wait()

            def rgrp(k, carry2):
                sl = pl.ds(16 * k, 16)
                srv = sbuf[sl] / (zbv[sl] + 1e-16)
                zv0 = 1.0 / (z0ch[sl] + 1e-16)
                zv1 = 1.0 / (z1ch[sl] + 1e-16)
                mv = xwbuf[sl]
                rv = scbuf[sl]
                for j in range(16):
                    r = 16 * k + j
                    sr = srv[j]
                    zi0 = zv0[j]
                    zi1 = zv1[j]
                    mean = mv[j]
                    rstd = rv[j]
                    for q in range(D // 16):
                        zi = zi0 if q < 4 else zi1
                        v = xbuf[r, pl.ds(16 * q, 16)] * zi + pbuf[2, pl.ds(16 * q, 16)]
                        wbuf[r, pl.ds(16 * q, 16)] = v * sr
                        ln = (v - mean) * rstd * pbuf[3, pl.ds(16 * q, 16)] + pbuf[4, pl.ds(16 * q, 16)]
                        xbuf[r, pl.ds(16 * q, 16)] = jnp.where(ln > 0, ln, jnp.exp(ln) - 1.0)
                return carry2

            lax.fori_loop(0, CH // 16, rgrp, 0)
            pltpu.sync_copy(wbuf, emb_sh.at[bidx], add=True)
            pltpu.sync_copy(xbuf, xnext_hbm.at[pl.ds(gbase, CH)])

        return carry

    lax.fori_loop(0, NIT, p4, 0)
    plsc.subcore_barrier()
    pltpu.sync_copy(emb_sh.at[pl.ds(s * GPT, GPT)],
                    emb_hbm.at[pl.ds(c * B + s * GPT, GPT)])


def _mm2_body(xr, wr, br, o0, o1):
    y = jnp.dot(xr[...], wr[...], preferred_element_type=jnp.float32) + br[...]
    o0[...] = y[:, 0:128]
    o1[...] = y[:, 128:256]


def _mm2(x, w, b, bm=2048):
    M, K = x.shape
    outs = [jax.ShapeDtypeStruct((M, D), jnp.float32) for _ in range(2)]
    return pl.pallas_call(
        _mm2_body,
        grid=(M // bm,),
        in_specs=[
            pl.BlockSpec((bm, K), lambda i: (i, 0)),
            pl.BlockSpec((K, 2 * D), lambda i: (0, 0)),
            pl.BlockSpec((1, 2 * D), lambda i: (0, 0)),
        ],
        out_specs=[pl.BlockSpec((bm, D), lambda i: (i, 0)) for _ in range(2)],
        out_shape=outs,
    )(x, w, b.reshape(1, 2 * D))


def _coattn_body(e1, e2, wq, wk, cb, ca, o):
    r1 = [e1[i] for i in range(NB)]
    r2 = [e2[i] for i in range(NB)]
    keys = [jnp.dot(r, wk[...], preferred_element_type=jnp.float32) for r in r1]
    qrys = [jnp.dot(r, wq[...], preferred_element_type=jnp.float32) for r in r2]
    bias = cb[...]
    av = ca[...].reshape(D // 2, 1)
    r1n = [r / jnp.maximum(jnp.sqrt(jnp.sum(r * r, axis=1, keepdims=True)), 1e-12) for r in r1]
    r2n = [r / jnp.maximum(jnp.sqrt(jnp.sum(r * r, axis=1, keepdims=True)), 1e-12) for r in r2]
    for i in range(NB):
        for j in range(NB):
            att = jnp.dot(jnp.tanh(qrys[j] + keys[i] + bias), av,
                          preferred_element_type=jnp.float32)
            o[:, pl.ds(D * (NB * i + j), D)] = (r1n[i] + r2n[j]) * att


def _coattn(embs, wq, wk, cb, ca):
    return pl.pallas_call(
        _coattn_body,
        grid=(1,),
        in_specs=[
            pl.BlockSpec((NB, B, D), lambda i: (0, 0, 0)),
            pl.BlockSpec((NB, B, D), lambda i: (0, 1, 0)),
            pl.BlockSpec((D, D // 2), lambda i: (0, 0)),
            pl.BlockSpec((D, D // 2), lambda i: (0, 0)),
            pl.BlockSpec((1, D // 2), lambda i: (0, 0)),
            pl.BlockSpec((1, D // 2), lambda i: (0, 0)),
        ],
        out_specs=pl.BlockSpec((B, HID), lambda i: (0, 0)),
        out_shape=jax.ShapeDtypeStruct((B, HID), jnp.float32),
    )(embs, embs, wq, wk, cb.reshape(1, D // 2), ca.reshape(1, D // 2))


def _mlp_layer_body(xr, wr, br, gr, ber, o):
    h = jnp.dot(xr[...], wr[...], preferred_element_type=jnp.float32) + br[...]
    mu = jnp.mean(h, axis=0, keepdims=True)
    xc = h - mu
    var = jnp.mean(xc * xc, axis=0, keepdims=True)
    h = xc / jnp.sqrt(var + 1e-5) * gr[...] + ber[...]
    o[...] = jnp.maximum(h, 0.0)


def _mlp_layer(x, w, b, g, be, bk=512):
    M, K = x.shape
    Ko = w.shape[1]
    return pl.pallas_call(
        _mlp_layer_body,
        grid=(Ko // bk,),
        in_specs=[
            pl.BlockSpec((M, K), lambda j: (0, 0)),
            pl.BlockSpec((K, bk), lambda j: (0, j)),
            pl.BlockSpec((1, bk), lambda j: (0, j)),
            pl.BlockSpec((1, bk), lambda j: (0, j)),
            pl.BlockSpec((1, bk), lambda j: (0, j)),
        ],
        out_specs=pl.BlockSpec((M, bk), lambda j: (0, j)),
        out_shape=jax.ShapeDtypeStruct((M, Ko), jnp.float32),
    )(x, w, b.reshape(1, Ko), g.reshape(1, Ko), be.reshape(1, Ko))


def _mm_body(xr, wr, br, o):
    o[...] = jnp.dot(xr[...], wr[...], preferred_element_type=jnp.float32) + br[...]


def _mm(x, w, b, bm=1024):
    M, K = x.shape
    Ko = w.shape[1]
    return pl.pallas_call(
        _mm_body,
        grid=(M // bm,),
        in_specs=[
            pl.BlockSpec((bm, K), lambda i: (i, 0)),
            pl.BlockSpec((K, Ko), lambda i: (0, 0)),
            pl.BlockSpec((1, Ko), lambda i: (0, 0)),
        ],
        out_specs=pl.BlockSpec((bm, Ko), lambda i: (i, 0)),
        out_shape=jax.ShapeDtypeStruct((M, Ko), jnp.float32),
    )(x, w, b.reshape(1, Ko))


def kernel(x1, edge_index1, x1_batch, x2, edge_index2, x2_batch, ln0_w, ln0_b,
           gat_Wl, gat_bl, gat_Wr, gat_br, gat_att, gat_bias, sag_Wrel, sag_brel,
           sag_Wroot, nn_w, nn_b, ca_wq, ca_wk, ca_bias, ca_a, mlp_Wh, mlp_bh,
           mlp_g, mlp_be, mlp_Wo, mlp_bo):
    x = jnp.concatenate([x1, x2], axis=0)
    batch = jnp.stack([x1_batch, x2_batch]).astype(jnp.int32)
    srcg = jnp.stack([edge_index1[0], edge_index2[0] + N]).astype(jnp.int32)
    dstg = jnp.stack([edge_index1[1], edge_index2[1] + N]).astype(jnp.int32)
    srcl = jnp.stack([edge_index1[0], edge_index2[0]]).astype(jnp.int32)
    dstl = jnp.stack([edge_index1[1], edge_index2[1]]).astype(jnp.int32)

    zrow = jnp.zeros((D,), jnp.float32)
    wb0 = jnp.stack([ln0_w, ln0_b] + [zrow] * 6)
    x = _sc_ln0(x, batch, wb0)

    embs = []
    for i in range(NB):
        wlr = jnp.concatenate([gat_Wl[i], gat_Wr[i]], axis=1)
        blr = jnp.concatenate([gat_bl[i], gat_br[i]], axis=0)
        xl, xr = _mm2(x, wlr, blr)
        attp = jnp.stack([jnp.pad(gat_att[i][0], (0, D - C)),
                          jnp.pad(gat_att[i][1], (0, D - C))] + [zrow] * 6)
        xgh, zz, _ = _sc_edge(xl, xr, srcg, dstg, dstl, attp)
        par = jnp.stack([
            sag_Wrel[i][:, 0],
            sag_Wroot[i][:, 0],
            gat_bias[i],
            nn_w[i],
            nn_b[i],
            jnp.full((D,), sag_brel[i][0], jnp.float32),
            zrow,
            zrow,
        ])
        x, emb = _sc_readout(xgh, zz, srcl, dstl, batch, par)
        embs.append(emb)

    embs = jnp.stack(embs, axis=0)
    h = _coattn(embs, ca_wq, ca_wk, ca_bias, ca_a)

    for l in range(3):
        h = _mlp_layer(h, mlp_Wh[l], mlp_bh[l], mlp_g[l], mlp_be[l])

    wo = jnp.concatenate([mlp_Wo, jnp.zeros((HID, 128 - OUT), jnp.float32)], axis=1)
    bo = jnp.concatenate([mlp_bo, jnp.zeros((128 - OUT,), jnp.float32)])
    out = _mm(h, wo, bo, bm=1024)
    return out[:, :OUT]
